# Initial kernel scaffold; baseline (speedup 1.0000x reference)
import jax
import jax.numpy as jnp
from jax.experimental import pallas as pl


def kernel(atomic_num, coord, length, abs_mask, edge_index, W_atom, b_atom, W_coord, b_coord, W_node, b_node, W_edge, b_edge, agg_W0, agg_b0, glu_Wv0, glu_bv0, glu_Wg0, glu_bg0, exp0, eps0, agg_W1, agg_b1, glu_Wv1, glu_bv1, glu_Wg1, glu_bg1, exp1, eps1, W_mlp, b_mlp):
    raise NotImplementedError("write your pallas kernel here")


# speedup vs baseline: 2.9784x; 2.9784x over previous
import functools

import jax
import jax.numpy as jnp
from jax import lax
from jax.experimental import pallas as pl
from jax.experimental.pallas import tpu as pltpu
from jax.experimental.pallas import tpu_sc as plsc

N = 10000
E = 320000
D = 128
ATOM_DIM = 118
OUT_DIM = 100

NC = 2
NS = 16
NW = NC * NS
EW = E // NW
CHUNK = 80
NCHUNK = EW // CHUNK
ROWS_PER_TILE = (N // NS) // 8 * 8
TAIL_ROWS = N - NS * ROWS_PER_TILE


_MESH = plsc.VectorSubcoreMesh(core_axis_name="c", subcore_axis_name="s")


def _sc_body(h_hbm, src_hbm, dst_hbm, w_hbm, z128_hbm, s_out,
             src_v, dst_v, w_v, rows_v, s_sh, sem):
    c = lax.axis_index("c")
    s = lax.axis_index("s")
    wid = c * NS + s
    r0 = s * ROWS_PER_TILE
    pltpu.sync_copy(z128_hbm.at[pl.ds(r0, ROWS_PER_TILE)],
                    s_sh.at[pl.ds(r0, ROWS_PER_TILE)])

    @pl.when(s == 0)
    def _zero_tail():
        t0 = NS * ROWS_PER_TILE
        pltpu.sync_copy(z128_hbm.at[pl.ds(t0, TAIL_ROWS)],
                        s_sh.at[pl.ds(t0, TAIL_ROWS)])

    plsc.subcore_barrier()

    base0 = wid * EW

    def chunk_body(i, carry):
        base = base0 + i * CHUNK
        pltpu.sync_copy(src_hbm.at[pl.ds(base, CHUNK)], src_v)
        pltpu.sync_copy(dst_hbm.at[pl.ds(base, CHUNK)], dst_v)
        pltpu.sync_copy(w_hbm.at[pl.ds(base, CHUNK)], w_v)
        pltpu.async_copy(h_hbm.at[src_v], rows_v, sem).wait()

        def row_body(r, carry2):
            wb = w_v[r]
            for cb in range(D // 16):
                sl = pl.ds(cb * 16, 16)
                rows_v[r, sl] = rows_v[r, sl] * wb
            return carry2

        lax.fori_loop(0, CHUNK, row_body, 0, unroll=4)
        pltpu.sync_copy(rows_v, s_sh.at[dst_v], add=True)
        return carry

    lax.fori_loop(0, NCHUNK, chunk_body, 0)
    plsc.subcore_barrier()
    pltpu.sync_copy(s_sh.at[pl.ds(r0, ROWS_PER_TILE)],
                    s_out.at[c].at[pl.ds(r0, ROWS_PER_TILE)])

    @pl.when(s == 0)
    def _out_tail():
        t0 = NS * ROWS_PER_TILE
        pltpu.sync_copy(s_sh.at[pl.ds(t0, TAIL_ROWS)],
                        s_out.at[c].at[pl.ds(t0, TAIL_ROWS)])


_sc_scatter = pl.kernel(
    _sc_body,
    out_type=jax.ShapeDtypeStruct((NC, N, D), jnp.float32),
    mesh=_MESH,
    scratch_types=[
        pltpu.VMEM((CHUNK,), jnp.int32),
        pltpu.VMEM((CHUNK,), jnp.int32),
        pltpu.VMEM((CHUNK, 16), jnp.float32),
        pltpu.VMEM((CHUNK, D), jnp.float32),
        pltpu.VMEM_SHARED((N, D), jnp.float32),
        pltpu.SemaphoreType.DMA,
    ],
)


def _sc_extras_body(dst_hbm, ext_hbm, z128_hbm, x_out,
                    dst_v, ext_v, rows_v, x_sh, sem):
    c = lax.axis_index("c")
    s = lax.axis_index("s")
    wid = c * NS + s
    r0 = s * ROWS_PER_TILE
    pltpu.sync_copy(z128_hbm.at[pl.ds(r0, ROWS_PER_TILE)],
                    x_sh.at[pl.ds(r0, ROWS_PER_TILE)])

    @pl.when(s == 0)
    def _zero_tail():
        t0 = NS * ROWS_PER_TILE
        pltpu.sync_copy(z128_hbm.at[pl.ds(t0, TAIL_ROWS)],
                        x_sh.at[pl.ds(t0, TAIL_ROWS)])

    zv = jnp.zeros((16,), jnp.float32)

    def zrow(r, carry):
        for cb in range(D // 16):
            rows_v[r, pl.ds(cb * 16, 16)] = zv
        return carry

    lax.fori_loop(0, CHUNK, zrow, 0, unroll=4)
    plsc.subcore_barrier()

    base0 = wid * EW

    def chunk_body(i, carry):
        base = base0 + i * CHUNK
        pltpu.sync_copy(dst_hbm.at[pl.ds(base, CHUNK)], dst_v)
        pltpu.sync_copy(ext_hbm.at[pl.ds(base, CHUNK)], ext_v)

        def row_body(r, carry2):
            rows_v[r, pl.ds(0, 16)] = ext_v[r]
            return carry2

        lax.fori_loop(0, CHUNK, row_body, 0, unroll=8)
        pltpu.sync_copy(rows_v, x_sh.at[dst_v], add=True)
        return carry

    lax.fori_loop(0, NCHUNK, chunk_body, 0)
    plsc.subcore_barrier()
    pltpu.sync_copy(x_sh.at[pl.ds(r0, ROWS_PER_TILE)],
                    x_out.at[c].at[pl.ds(r0, ROWS_PER_TILE)])

    @pl.when(s == 0)
    def _out_tail():
        t0 = NS * ROWS_PER_TILE
        pltpu.sync_copy(x_sh.at[pl.ds(t0, TAIL_ROWS)],
                        x_out.at[c].at[pl.ds(t0, TAIL_ROWS)])


_sc_extras = pl.kernel(
    _sc_extras_body,
    out_type=jax.ShapeDtypeStruct((NC, N, D), jnp.float32),
    mesh=_MESH,
    scratch_types=[
        pltpu.VMEM((CHUNK,), jnp.int32),
        pltpu.VMEM((CHUNK, 16), jnp.float32),
        pltpu.VMEM((CHUNK, D), jnp.float32),
        pltpu.VMEM_SHARED((N, D), jnp.float32),
        pltpu.SemaphoreType.DMA,
    ],
)


def _embed_body(an_ref, co_ref, wa_ref, ba_ref, wc_ref, bc_ref, wn_ref,
                bn_ref, h_ref):
    a = jnp.dot(an_ref[:], wa_ref[:], preferred_element_type=jnp.float32)
    a = a + ba_ref[:]
    c = jnp.dot(co_ref[:], wc_ref[:], preferred_element_type=jnp.float32)
    c = c + bc_ref[:]
    ac = jnp.concatenate([a, c], axis=1)
    h_ref[:] = jnp.dot(ac, wn_ref[:], preferred_element_type=jnp.float32) + bn_ref[:]


def _embed(atomic_num, coord, W_atom, b_atom, W_coord, b_coord, W_node, b_node):
    return pl.pallas_call(
        _embed_body,
        out_shape=jax.ShapeDtypeStruct((N, D), jnp.float32),
    )(atomic_num, coord, W_atom, b_atom.reshape(1, D),
      W_coord, b_coord.reshape(1, D), W_node, b_node.reshape(1, D))


_EB = 10000


def _edgew_body(len_ref, e0_ref, e1_ref, w0_ref, w1_ref, ext_ref):
    ln = len_ref[:]
    lg = jnp.log(ln)
    w0 = jnp.exp(lg * e0_ref[0, 0])
    w1 = jnp.exp(lg * e1_ref[0, 0])
    w0_ref[:] = jnp.broadcast_to(w0, (_EB, 16))
    w1_ref[:] = jnp.broadcast_to(w1, (_EB, 16))
    z = jnp.zeros((_EB, 12), jnp.float32)
    ext_ref[:] = jnp.concatenate([w0, w0 * ln, w1, w1 * ln, z], axis=1)


def _edgew(length, exp0, exp1):
    nb = E // _EB
    return pl.pallas_call(
        _edgew_body,
        grid=(nb,),
        in_specs=[
            pl.BlockSpec((_EB, 1), lambda i: (i, 0)),
            pl.BlockSpec((1, 1), lambda i: (0, 0)),
            pl.BlockSpec((1, 1), lambda i: (0, 0)),
        ],
        out_specs=[
            pl.BlockSpec((_EB, 16), lambda i: (i, 0)),
            pl.BlockSpec((_EB, 16), lambda i: (i, 0)),
            pl.BlockSpec((_EB, 16), lambda i: (i, 0)),
        ],
        out_shape=[
            jax.ShapeDtypeStruct((E, 16), jnp.float32),
            jax.ShapeDtypeStruct((E, 16), jnp.float32),
            jax.ShapeDtypeStruct((E, 16), jnp.float32),
        ],
    )(length, exp0.reshape(1, 1), exp1.reshape(1, 1))


_NB = 1000
_NGRID = N // _NB


def _conv_tail(s2, x2, h, agg_w, agg_b, w_edge, b_edge, wv, bv, wg, bg, eps,
               s_cols):
    S = s2[0] + s2[1]
    X = x2[0] + x2[1]
    ws = agg_w[0:D, :]
    we = agg_w[D:2 * D, :]
    wd = agg_w[2 * D:3 * D, :]
    u = jnp.dot(w_edge, we, preferred_element_type=jnp.float32)
    ce = jnp.dot(b_edge, we, preferred_element_type=jnp.float32) + agg_b
    s0 = X[:, s_cols[0]:s_cols[0] + 1]
    s1 = X[:, s_cols[1]:s_cols[1] + 1]
    agg = (jnp.dot(S, ws, preferred_element_type=jnp.float32)
           + s1 * u + s0 * ce
           + jnp.dot(s0 * h, wd, preferred_element_type=jnp.float32))
    rst = (1.0 + eps) * h + agg
    gv = jnp.dot(rst, wv, preferred_element_type=jnp.float32) + bv
    gg = jnp.dot(rst, wg, preferred_element_type=jnp.float32) + bg
    return gv * jax.nn.sigmoid(gg)


def _combine_mid_body(s2_ref, x2_ref, h_ref, aggw_ref, aggb_ref, wed_ref,
                      bed_ref, wv_ref, bv_ref, wg_ref, bg_ref, eps_ref,
                      h1_ref):
    h1_ref[:] = _conv_tail(
        s2_ref[:], x2_ref[:], h_ref[:], aggw_ref[:], aggb_ref[:], wed_ref[:],
        bed_ref[:], wv_ref[:], bv_ref[:], wg_ref[:], bg_ref[:],
        eps_ref[0, 0], (0, 1))


def _combine_mid(s2, x2, h, agg_w, agg_b, w_edge, b_edge, wv, bv, wg, bg, eps):
    full = lambda shape: pl.BlockSpec(shape, lambda i: tuple(0 for _ in shape))
    return pl.pallas_call(
        _combine_mid_body,
        grid=(_NGRID,),
        in_specs=[
            pl.BlockSpec((NC, _NB, D), lambda i: (0, i, 0)),
            pl.BlockSpec((NC, _NB, D), lambda i: (0, i, 0)),
            pl.BlockSpec((_NB, D), lambda i: (i, 0)),
            full((3 * D, D)), full((1, D)), full((1, D)), full((1, D)),
            full((D, D)), full((1, D)), full((D, D)), full((1, D)),
            full((1, 1)),
        ],
        out_specs=pl.BlockSpec((_NB, D), lambda i: (i, 0)),
        out_shape=jax.ShapeDtypeStruct((N, D), jnp.float32),
    )(s2, x2, h, agg_w, agg_b.reshape(1, D), w_edge, b_edge.reshape(1, D),
      wv, bv.reshape(1, D), wg, bg.reshape(1, D), eps.reshape(1, 1))


def _combine_last_body(s2_ref, x2_ref, h_ref, aggw_ref, aggb_ref, wed_ref,
                       bed_ref, wv_ref, bv_ref, wg_ref, bg_ref, eps_ref,
                       mask_ref, feat_ref):
    h2 = _conv_tail(
        s2_ref[:], x2_ref[:], h_ref[:], aggw_ref[:], aggb_ref[:], wed_ref[:],
        bed_ref[:], wv_ref[:], bv_ref[:], wg_ref[:], bg_ref[:],
        eps_ref[0, 0], (2, 3))
    h2 = jnp.where(mask_ref[:] == 0, 0.0, h2)
    part = jnp.sum(h2, axis=0, keepdims=True) * (1.0 / N)

    @pl.when(pl.program_id(0) == 0)
    def _init():
        feat_ref[:] = jnp.zeros_like(feat_ref)

    feat_ref[:] += part


def _combine_last(s2, x2, h, agg_w, agg_b, w_edge, b_edge, wv, bv, wg, bg,
                  eps, abs_mask):
    full = lambda shape: pl.BlockSpec(shape, lambda i: tuple(0 for _ in shape))
    return pl.pallas_call(
        _combine_last_body,
        grid=(_NGRID,),
        in_specs=[
            pl.BlockSpec((NC, _NB, D), lambda i: (0, i, 0)),
            pl.BlockSpec((NC, _NB, D), lambda i: (0, i, 0)),
            pl.BlockSpec((_NB, D), lambda i: (i, 0)),
            full((3 * D, D)), full((1, D)), full((1, D)), full((1, D)),
            full((D, D)), full((1, D)), full((D, D)), full((1, D)),
            full((1, 1)),
            pl.BlockSpec((_NB, 1), lambda i: (i, 0)),
        ],
        out_specs=pl.BlockSpec((1, D), lambda i: (0, 0)),
        out_shape=jax.ShapeDtypeStruct((1, D), jnp.float32),
    )(s2, x2, h, agg_w, agg_b.reshape(1, D), w_edge, b_edge.reshape(1, D),
      wv, bv.reshape(1, D), wg, bg.reshape(1, D), eps.reshape(1, 1),
      abs_mask.reshape(N, 1))


def _head_body(feat_ref, wm_ref, bm_ref, out_ref):
    out_ref[:] = jax.nn.sigmoid(
        jnp.dot(feat_ref[:], wm_ref[:], preferred_element_type=jnp.float32)
        + bm_ref[:])


def _head(feat, W_mlp, b_mlp):
    return pl.pallas_call(
        _head_body,
        out_shape=jax.ShapeDtypeStruct((1, OUT_DIM), jnp.float32),
    )(feat, W_mlp, b_mlp.reshape(1, OUT_DIM))


def kernel(atomic_num, coord, length, abs_mask, edge_index, W_atom, b_atom,
           W_coord, b_coord, W_node, b_node, W_edge, b_edge, agg_W0, agg_b0,
           glu_Wv0, glu_bv0, glu_Wg0, glu_bg0, exp0, eps0, agg_W1, agg_b1,
           glu_Wv1, glu_bv1, glu_Wg1, glu_bg1, exp1, eps1, W_mlp, b_mlp):
    src = edge_index[0]
    dst = edge_index[1]
    h0 = _embed(atomic_num, coord, W_atom, b_atom, W_coord, b_coord,
                W_node, b_node)
    w0b, w1b, wext = _edgew(length, exp0, exp1)
    z128 = jnp.zeros((N, D), jnp.float32)

    s_l0 = _sc_scatter(h0, src, dst, w0b, z128)
    x_l0 = _sc_extras(dst, wext, z128)
    h1 = _combine_mid(s_l0, x_l0, h0, agg_W0, agg_b0, W_edge, b_edge,
                      glu_Wv0, glu_bv0, glu_Wg0, glu_bg0, eps0)
    s_l1 = _sc_scatter(h1, src, dst, w1b, z128)
    feat = _combine_last(s_l1, x_l0, h1, agg_W1, agg_b1, W_edge, b_edge,
                         glu_Wv1, glu_bv1, glu_Wg1, glu_bg1, eps1, abs_mask)
    return _head(feat, W_mlp, b_mlp)


# speedup vs baseline: 3.7890x; 1.2721x over previous
import functools

import jax
import jax.numpy as jnp
from jax import lax
from jax.experimental import pallas as pl
from jax.experimental.pallas import tpu as pltpu
from jax.experimental.pallas import tpu_sc as plsc

N = 10000
E = 320000
D = 128
ATOM_DIM = 118
OUT_DIM = 100

NC = 2
NS = 16
NW = NC * NS
EW = E // NW
CHUNK = 80
NCHUNK = EW // CHUNK
ROWS_PER_TILE = (N // NS) // 8 * 8
TAIL_ROWS = N - NS * ROWS_PER_TILE


_MESH = plsc.VectorSubcoreMesh(core_axis_name="c", subcore_axis_name="s")


def _sc_body(h_hbm, src_hbm, dst_hbm, w_hbm, z128_hbm, s_out,
             src_a, dst_a, w_a, rows_a, src_b, dst_b, w_b, rows_b,
             s_sh, g_a, g_b, s_a, s_b):
    c = lax.axis_index("c")
    s = lax.axis_index("s")
    wid = c * NS + s
    r0 = s * ROWS_PER_TILE
    pltpu.sync_copy(z128_hbm.at[pl.ds(r0, ROWS_PER_TILE)],
                    s_sh.at[pl.ds(r0, ROWS_PER_TILE)])

    @pl.when(s == 0)
    def _zero_tail():
        t0 = NS * ROWS_PER_TILE
        pltpu.sync_copy(z128_hbm.at[pl.ds(t0, TAIL_ROWS)],
                        s_sh.at[pl.ds(t0, TAIL_ROWS)])

    plsc.subcore_barrier()

    base0 = wid * EW

    def fetch(ci, src_v, dst_v, w_v, rows_v, g_sem):
        base = base0 + ci * CHUNK
        pltpu.sync_copy(src_hbm.at[pl.ds(base, CHUNK)], src_v)
        pltpu.sync_copy(dst_hbm.at[pl.ds(base, CHUNK)], dst_v)
        pltpu.sync_copy(w_hbm.at[pl.ds(base, CHUNK)], w_v)
        pltpu.async_copy(h_hbm.at[src_v], rows_v, g_sem)

    def scale(w_v, rows_v):
        def row_body(r, carry2):
            wb = w_v[r]
            for cb in range(D // 16):
                sl = pl.ds(cb * 16, 16)
                rows_v[r, sl] = rows_v[r, sl] * wb
            return carry2

        lax.fori_loop(0, CHUNK, row_body, 0, unroll=4)

    def wait_g(src_v, rows_v, g_sem):
        pltpu.make_async_copy(h_hbm.at[src_v], rows_v, g_sem).wait()

    def wait_s(dst_v, rows_v, s_sem):
        pltpu.make_async_copy(rows_v, s_sh.at[dst_v], s_sem).wait()

    fetch(0, src_a, dst_a, w_a, rows_a, g_a)

    def pair(i, carry):
        @pl.when(i > 0)
        def _():
            wait_s(dst_b, rows_b, s_b)
        fetch(2 * i + 1, src_b, dst_b, w_b, rows_b, g_b)
        wait_g(src_a, rows_a, g_a)
        scale(w_a, rows_a)
        pltpu.async_copy(rows_a, s_sh.at[dst_a], s_a, add=True)
        wait_g(src_b, rows_b, g_b)
        scale(w_b, rows_b)
        pltpu.async_copy(rows_b, s_sh.at[dst_b], s_b, add=True)
        wait_s(dst_a, rows_a, s_a)
        fetch(2 * i + 2, src_a, dst_a, w_a, rows_a, g_a)
        return carry

    lax.fori_loop(0, (NCHUNK - 1) // 2, pair, 0)
    wait_s(dst_b, rows_b, s_b)
    wait_g(src_a, rows_a, g_a)
    scale(w_a, rows_a)
    pltpu.sync_copy(rows_a, s_sh.at[dst_a], add=True)
    plsc.subcore_barrier()
    pltpu.sync_copy(s_sh.at[pl.ds(r0, ROWS_PER_TILE)],
                    s_out.at[c].at[pl.ds(r0, ROWS_PER_TILE)])

    @pl.when(s == 0)
    def _out_tail():
        t0 = NS * ROWS_PER_TILE
        pltpu.sync_copy(s_sh.at[pl.ds(t0, TAIL_ROWS)],
                        s_out.at[c].at[pl.ds(t0, TAIL_ROWS)])


_sc_scatter = pl.kernel(
    _sc_body,
    out_type=jax.ShapeDtypeStruct((NC, N, D), jnp.float32),
    mesh=_MESH,
    scratch_types=[
        pltpu.VMEM((CHUNK,), jnp.int32),
        pltpu.VMEM((CHUNK,), jnp.int32),
        pltpu.VMEM((CHUNK, 16), jnp.float32),
        pltpu.VMEM((CHUNK, D), jnp.float32),
        pltpu.VMEM((CHUNK,), jnp.int32),
        pltpu.VMEM((CHUNK,), jnp.int32),
        pltpu.VMEM((CHUNK, 16), jnp.float32),
        pltpu.VMEM((CHUNK, D), jnp.float32),
        pltpu.VMEM_SHARED((N, D), jnp.float32),
        pltpu.SemaphoreType.DMA,
        pltpu.SemaphoreType.DMA,
        pltpu.SemaphoreType.DMA,
        pltpu.SemaphoreType.DMA,
    ],
)


def _sc_extras_body(dst_hbm, ext_hbm, z128_hbm, x_out,
                    dst_a, ext_a, rows_a, dst_b, ext_b, rows_b, x_sh,
                    s_a, s_b):
    c = lax.axis_index("c")
    s = lax.axis_index("s")
    wid = c * NS + s
    r0 = s * ROWS_PER_TILE
    pltpu.sync_copy(z128_hbm.at[pl.ds(r0, ROWS_PER_TILE)],
                    x_sh.at[pl.ds(r0, ROWS_PER_TILE)])

    @pl.when(s == 0)
    def _zero_tail():
        t0 = NS * ROWS_PER_TILE
        pltpu.sync_copy(z128_hbm.at[pl.ds(t0, TAIL_ROWS)],
                        x_sh.at[pl.ds(t0, TAIL_ROWS)])

    zv = jnp.zeros((16,), jnp.float32)

    def zrow(r, carry):
        for cb in range(D // 16):
            rows_a[r, pl.ds(cb * 16, 16)] = zv
            rows_b[r, pl.ds(cb * 16, 16)] = zv
        return carry

    lax.fori_loop(0, CHUNK, zrow, 0, unroll=4)
    plsc.subcore_barrier()

    base0 = wid * EW

    def do_chunk(ci, dst_v, ext_v, rows_v, s_sem):
        base = base0 + ci * CHUNK
        pltpu.sync_copy(dst_hbm.at[pl.ds(base, CHUNK)], dst_v)
        pltpu.sync_copy(ext_hbm.at[pl.ds(base, CHUNK)], ext_v)

        def row_body(r, carry2):
            rows_v[r, pl.ds(0, 16)] = ext_v[r]
            return carry2

        lax.fori_loop(0, CHUNK, row_body, 0, unroll=8)
        pltpu.async_copy(rows_v, x_sh.at[dst_v], s_sem, add=True)

    def wait_s(dst_v, rows_v, s_sem):
        pltpu.make_async_copy(rows_v, x_sh.at[dst_v], s_sem).wait()

    def pair(i, carry):
        @pl.when(i > 0)
        def _():
            wait_s(dst_a, rows_a, s_a)
        do_chunk(2 * i, dst_a, ext_a, rows_a, s_a)

        @pl.when(i > 0)
        def _():
            wait_s(dst_b, rows_b, s_b)
        do_chunk(2 * i + 1, dst_b, ext_b, rows_b, s_b)
        return carry

    lax.fori_loop(0, (NCHUNK - 1) // 2, pair, 0)
    wait_s(dst_a, rows_a, s_a)
    do_chunk(NCHUNK - 1, dst_a, ext_a, rows_a, s_a)
    wait_s(dst_a, rows_a, s_a)
    wait_s(dst_b, rows_b, s_b)
    plsc.subcore_barrier()
    pltpu.sync_copy(x_sh.at[pl.ds(r0, ROWS_PER_TILE)],
                    x_out.at[c].at[pl.ds(r0, ROWS_PER_TILE)])

    @pl.when(s == 0)
    def _out_tail():
        t0 = NS * ROWS_PER_TILE
        pltpu.sync_copy(x_sh.at[pl.ds(t0, TAIL_ROWS)],
                        x_out.at[c].at[pl.ds(t0, TAIL_ROWS)])


_sc_extras = pl.kernel(
    _sc_extras_body,
    out_type=jax.ShapeDtypeStruct((NC, N, D), jnp.float32),
    mesh=_MESH,
    scratch_types=[
        pltpu.VMEM((CHUNK,), jnp.int32),
        pltpu.VMEM((CHUNK, 16), jnp.float32),
        pltpu.VMEM((CHUNK, D), jnp.float32),
        pltpu.VMEM((CHUNK,), jnp.int32),
        pltpu.VMEM((CHUNK, 16), jnp.float32),
        pltpu.VMEM((CHUNK, D), jnp.float32),
        pltpu.VMEM_SHARED((N, D), jnp.float32),
        pltpu.SemaphoreType.DMA,
        pltpu.SemaphoreType.DMA,
    ],
)


def _embed_body(an_ref, co_ref, wa_ref, ba_ref, wc_ref, bc_ref, wn_ref,
                bn_ref, h_ref):
    a = jnp.dot(an_ref[:], wa_ref[:], preferred_element_type=jnp.float32)
    a = a + ba_ref[:]
    c = jnp.dot(co_ref[:], wc_ref[:], preferred_element_type=jnp.float32)
    c = c + bc_ref[:]
    ac = jnp.concatenate([a, c], axis=1)
    h_ref[:] = jnp.dot(ac, wn_ref[:], preferred_element_type=jnp.float32) + bn_ref[:]


def _embed(atomic_num, coord, W_atom, b_atom, W_coord, b_coord, W_node, b_node):
    return pl.pallas_call(
        _embed_body,
        out_shape=jax.ShapeDtypeStruct((N, D), jnp.float32),
    )(atomic_num, coord, W_atom, b_atom.reshape(1, D),
      W_coord, b_coord.reshape(1, D), W_node, b_node.reshape(1, D))


_EB = 10000


def _edgew_body(len_ref, e0_ref, e1_ref, w0_ref, w1_ref, ext_ref):
    ln = len_ref[:]
    lg = jnp.log(ln)
    w0 = jnp.exp(lg * e0_ref[0, 0])
    w1 = jnp.exp(lg * e1_ref[0, 0])
    w0_ref[:] = jnp.broadcast_to(w0, (_EB, 16))
    w1_ref[:] = jnp.broadcast_to(w1, (_EB, 16))
    z = jnp.zeros((_EB, 12), jnp.float32)
    ext_ref[:] = jnp.concatenate([w0, w0 * ln, w1, w1 * ln, z], axis=1)


def _edgew(length, exp0, exp1):
    nb = E // _EB
    return pl.pallas_call(
        _edgew_body,
        grid=(nb,),
        in_specs=[
            pl.BlockSpec((_EB, 1), lambda i: (i, 0)),
            pl.BlockSpec((1, 1), lambda i: (0, 0)),
            pl.BlockSpec((1, 1), lambda i: (0, 0)),
        ],
        out_specs=[
            pl.BlockSpec((_EB, 16), lambda i: (i, 0)),
            pl.BlockSpec((_EB, 16), lambda i: (i, 0)),
            pl.BlockSpec((_EB, 16), lambda i: (i, 0)),
        ],
        out_shape=[
            jax.ShapeDtypeStruct((E, 16), jnp.float32),
            jax.ShapeDtypeStruct((E, 16), jnp.float32),
            jax.ShapeDtypeStruct((E, 16), jnp.float32),
        ],
    )(length, exp0.reshape(1, 1), exp1.reshape(1, 1))


_NB = 1000
_NGRID = N // _NB


def _conv_tail(s2, x2, h, agg_w, agg_b, w_edge, b_edge, wv, bv, wg, bg, eps,
               s_cols):
    S = s2[0] + s2[1]
    X = x2[0] + x2[1]
    ws = agg_w[0:D, :]
    we = agg_w[D:2 * D, :]
    wd = agg_w[2 * D:3 * D, :]
    u = jnp.dot(w_edge, we, preferred_element_type=jnp.float32)
    ce = jnp.dot(b_edge, we, preferred_element_type=jnp.float32) + agg_b
    s0 = X[:, s_cols[0]:s_cols[0] + 1]
    s1 = X[:, s_cols[1]:s_cols[1] + 1]
    agg = (jnp.dot(S, ws, preferred_element_type=jnp.float32)
           + s1 * u + s0 * ce
           + jnp.dot(s0 * h, wd, preferred_element_type=jnp.float32))
    rst = (1.0 + eps) * h + agg
    gv = jnp.dot(rst, wv, preferred_element_type=jnp.float32) + bv
    gg = jnp.dot(rst, wg, preferred_element_type=jnp.float32) + bg
    return gv * jax.nn.sigmoid(gg)


def _combine_mid_body(s2_ref, x2_ref, h_ref, aggw_ref, aggb_ref, wed_ref,
                      bed_ref, wv_ref, bv_ref, wg_ref, bg_ref, eps_ref,
                      h1_ref):
    h1_ref[:] = _conv_tail(
        s2_ref[:], x2_ref[:], h_ref[:], aggw_ref[:], aggb_ref[:], wed_ref[:],
        bed_ref[:], wv_ref[:], bv_ref[:], wg_ref[:], bg_ref[:],
        eps_ref[0, 0], (0, 1))


def _combine_mid(s2, x2, h, agg_w, agg_b, w_edge, b_edge, wv, bv, wg, bg, eps):
    full = lambda shape: pl.BlockSpec(shape, lambda i: tuple(0 for _ in shape))
    return pl.pallas_call(
        _combine_mid_body,
        grid=(_NGRID,),
        in_specs=[
            pl.BlockSpec((NC, _NB, D), lambda i: (0, i, 0)),
            pl.BlockSpec((NC, _NB, D), lambda i: (0, i, 0)),
            pl.BlockSpec((_NB, D), lambda i: (i, 0)),
            full((3 * D, D)), full((1, D)), full((1, D)), full((1, D)),
            full((D, D)), full((1, D)), full((D, D)), full((1, D)),
            full((1, 1)),
        ],
        out_specs=pl.BlockSpec((_NB, D), lambda i: (i, 0)),
        out_shape=jax.ShapeDtypeStruct((N, D), jnp.float32),
    )(s2, x2, h, agg_w, agg_b.reshape(1, D), w_edge, b_edge.reshape(1, D),
      wv, bv.reshape(1, D), wg, bg.reshape(1, D), eps.reshape(1, 1))


def _combine_last_body(s2_ref, x2_ref, h_ref, aggw_ref, aggb_ref, wed_ref,
                       bed_ref, wv_ref, bv_ref, wg_ref, bg_ref, eps_ref,
                       mask_ref, feat_ref):
    h2 = _conv_tail(
        s2_ref[:], x2_ref[:], h_ref[:], aggw_ref[:], aggb_ref[:], wed_ref[:],
        bed_ref[:], wv_ref[:], bv_ref[:], wg_ref[:], bg_ref[:],
        eps_ref[0, 0], (2, 3))
    h2 = jnp.where(mask_ref[:] == 0, 0.0, h2)
    part = jnp.sum(h2, axis=0, keepdims=True) * (1.0 / N)

    @pl.when(pl.program_id(0) == 0)
    def _init():
        feat_ref[:] = jnp.zeros_like(feat_ref)

    feat_ref[:] += part


def _combine_last(s2, x2, h, agg_w, agg_b, w_edge, b_edge, wv, bv, wg, bg,
                  eps, abs_mask):
    full = lambda shape: pl.BlockSpec(shape, lambda i: tuple(0 for _ in shape))
    return pl.pallas_call(
        _combine_last_body,
        grid=(_NGRID,),
        in_specs=[
            pl.BlockSpec((NC, _NB, D), lambda i: (0, i, 0)),
            pl.BlockSpec((NC, _NB, D), lambda i: (0, i, 0)),
            pl.BlockSpec((_NB, D), lambda i: (i, 0)),
            full((3 * D, D)), full((1, D)), full((1, D)), full((1, D)),
            full((D, D)), full((1, D)), full((D, D)), full((1, D)),
            full((1, 1)),
            pl.BlockSpec((_NB, 1), lambda i: (i, 0)),
        ],
        out_specs=pl.BlockSpec((1, D), lambda i: (0, 0)),
        out_shape=jax.ShapeDtypeStruct((1, D), jnp.float32),
    )(s2, x2, h, agg_w, agg_b.reshape(1, D), w_edge, b_edge.reshape(1, D),
      wv, bv.reshape(1, D), wg, bg.reshape(1, D), eps.reshape(1, 1),
      abs_mask.reshape(N, 1))


def _head_body(feat_ref, wm_ref, bm_ref, out_ref):
    out_ref[:] = jax.nn.sigmoid(
        jnp.dot(feat_ref[:], wm_ref[:], preferred_element_type=jnp.float32)
        + bm_ref[:])


def _head(feat, W_mlp, b_mlp):
    return pl.pallas_call(
        _head_body,
        out_shape=jax.ShapeDtypeStruct((1, OUT_DIM), jnp.float32),
    )(feat, W_mlp, b_mlp.reshape(1, OUT_DIM))


def kernel(atomic_num, coord, length, abs_mask, edge_index, W_atom, b_atom,
           W_coord, b_coord, W_node, b_node, W_edge, b_edge, agg_W0, agg_b0,
           glu_Wv0, glu_bv0, glu_Wg0, glu_bg0, exp0, eps0, agg_W1, agg_b1,
           glu_Wv1, glu_bv1, glu_Wg1, glu_bg1, exp1, eps1, W_mlp, b_mlp):
    src = edge_index[0]
    dst = edge_index[1]
    h0 = _embed(atomic_num, coord, W_atom, b_atom, W_coord, b_coord,
                W_node, b_node)
    w0b, w1b, wext = _edgew(length, exp0, exp1)
    z128 = jnp.zeros((N, D), jnp.float32)

    s_l0 = _sc_scatter(h0, src, dst, w0b, z128)
    x_l0 = _sc_extras(dst, wext, z128)
    h1 = _combine_mid(s_l0, x_l0, h0, agg_W0, agg_b0, W_edge, b_edge,
                      glu_Wv0, glu_bv0, glu_Wg0, glu_bg0, eps0)
    s_l1 = _sc_scatter(h1, src, dst, w1b, z128)
    feat = _combine_last(s_l1, x_l0, h1, agg_W1, agg_b1, W_edge, b_edge,
                         glu_Wv1, glu_bv1, glu_Wg1, glu_bg1, eps1, abs_mask)
    return _head(feat, W_mlp, b_mlp)


# speedup vs baseline: 4.0432x; 1.0671x over previous
import functools

import jax
import jax.numpy as jnp
from jax import lax
from jax.experimental import pallas as pl
from jax.experimental.pallas import tpu as pltpu
from jax.experimental.pallas import tpu_sc as plsc

N = 10000
E = 320000
D = 128
ATOM_DIM = 118
OUT_DIM = 100

NC = 2
NS = 16
NW = NC * NS
EW = E // NW
CHUNK = 80
NCHUNK = -(-EW // CHUNK)
EWP = NCHUNK * CHUNK
EPAD = NW * EWP
ROWS_PER_TILE = (N // NS) // 8 * 8
TAIL_ROWS = N - NS * ROWS_PER_TILE


_MESH = plsc.VectorSubcoreMesh(core_axis_name="c", subcore_axis_name="s")


def _sc_body(h_hbm, p3_hbm, w_hbm, z128_hbm, s_out,
             pp_a, w_a, rows_a, pp_b, w_b, rows_b,
             s_sh, g_a, g_b, s_a, s_b):
    c = lax.axis_index("c")
    s = lax.axis_index("s")
    wid = c * NS + s
    r0 = s * ROWS_PER_TILE
    pltpu.sync_copy(z128_hbm.at[pl.ds(r0, ROWS_PER_TILE)],
                    s_sh.at[pl.ds(r0, ROWS_PER_TILE)])

    @pl.when(s == 0)
    def _zero_tail():
        t0 = NS * ROWS_PER_TILE
        pltpu.sync_copy(z128_hbm.at[pl.ds(t0, TAIL_ROWS)],
                        s_sh.at[pl.ds(t0, TAIL_ROWS)])

    plsc.subcore_barrier()

    cbase = wid * NCHUNK

    def fetch(ci, pp_v, w_v, rows_v, g_sem):
        pltpu.sync_copy(p3_hbm.at[cbase + ci], pp_v)
        pltpu.sync_copy(w_hbm.at[pl.ds((cbase + ci) * CHUNK, CHUNK)], w_v)
        pltpu.async_copy(h_hbm.at[pp_v.at[0]], rows_v, g_sem)

    def scale(w_v, rows_v):
        def row_body(r, carry2):
            wb = w_v[r]
            for cb in range(D // 16):
                sl = pl.ds(cb * 16, 16)
                rows_v[r, sl] = rows_v[r, sl] * wb
            return carry2

        lax.fori_loop(0, CHUNK, row_body, 0, unroll=4)

    def wait_g(pp_v, rows_v, g_sem):
        pltpu.make_async_copy(h_hbm.at[pp_v.at[0]], rows_v, g_sem).wait()

    def wait_s(pp_v, rows_v, s_sem):
        pltpu.make_async_copy(rows_v, s_sh.at[pp_v.at[1]], s_sem).wait()

    fetch(0, pp_a, w_a, rows_a, g_a)

    def pair(i, carry):
        @pl.when(i > 0)
        def _():
            wait_s(pp_b, rows_b, s_b)
        fetch(2 * i + 1, pp_b, w_b, rows_b, g_b)
        wait_g(pp_a, rows_a, g_a)
        scale(w_a, rows_a)
        pltpu.async_copy(rows_a, s_sh.at[pp_a.at[1]], s_a, add=True)
        wait_g(pp_b, rows_b, g_b)
        scale(w_b, rows_b)
        pltpu.async_copy(rows_b, s_sh.at[pp_b.at[1]], s_b, add=True)
        wait_s(pp_a, rows_a, s_a)
        fetch(2 * i + 2, pp_a, w_a, rows_a, g_a)
        return carry

    lax.fori_loop(0, (NCHUNK - 1) // 2, pair, 0)
    wait_s(pp_b, rows_b, s_b)
    wait_g(pp_a, rows_a, g_a)
    scale(w_a, rows_a)
    pltpu.sync_copy(rows_a, s_sh.at[pp_a.at[1]], add=True)
    plsc.subcore_barrier()
    pltpu.sync_copy(s_sh.at[pl.ds(r0, ROWS_PER_TILE)],
                    s_out.at[c].at[pl.ds(r0, ROWS_PER_TILE)])

    @pl.when(s == 0)
    def _out_tail():
        t0 = NS * ROWS_PER_TILE
        pltpu.sync_copy(s_sh.at[pl.ds(t0, TAIL_ROWS)],
                        s_out.at[c].at[pl.ds(t0, TAIL_ROWS)])


_sc_scatter = pl.kernel(
    _sc_body,
    out_type=jax.ShapeDtypeStruct((NC, N, D), jnp.float32),
    mesh=_MESH,
    scratch_types=[
        pltpu.VMEM((2, CHUNK), jnp.int32),
        pltpu.VMEM((CHUNK, 16), jnp.float32),
        pltpu.VMEM((CHUNK, D), jnp.float32),
        pltpu.VMEM((2, CHUNK), jnp.int32),
        pltpu.VMEM((CHUNK, 16), jnp.float32),
        pltpu.VMEM((CHUNK, D), jnp.float32),
        pltpu.VMEM_SHARED((N, D), jnp.float32),
        pltpu.SemaphoreType.DMA,
        pltpu.SemaphoreType.DMA,
        pltpu.SemaphoreType.DMA,
        pltpu.SemaphoreType.DMA,
    ],
)


def _sc_extras_body(dst_hbm, ext_hbm, z128_hbm, x_out,
                    dst_a, ext_a, rows_a, dst_b, ext_b, rows_b, x_sh,
                    s_a, s_b):
    c = lax.axis_index("c")
    s = lax.axis_index("s")
    wid = c * NS + s
    r0 = s * ROWS_PER_TILE
    pltpu.sync_copy(z128_hbm.at[pl.ds(r0, ROWS_PER_TILE)],
                    x_sh.at[pl.ds(r0, ROWS_PER_TILE)])

    @pl.when(s == 0)
    def _zero_tail():
        t0 = NS * ROWS_PER_TILE
        pltpu.sync_copy(z128_hbm.at[pl.ds(t0, TAIL_ROWS)],
                        x_sh.at[pl.ds(t0, TAIL_ROWS)])

    zv = jnp.zeros((16,), jnp.float32)

    def zrow(r, carry):
        for cb in range(D // 16):
            rows_a[r, pl.ds(cb * 16, 16)] = zv
            rows_b[r, pl.ds(cb * 16, 16)] = zv
        return carry

    lax.fori_loop(0, CHUNK, zrow, 0, unroll=4)
    plsc.subcore_barrier()

    base0 = wid * EWP

    def do_chunk(ci, dst_v, ext_v, rows_v, s_sem):
        base = base0 + ci * CHUNK
        pltpu.sync_copy(dst_hbm.at[pl.ds(base, CHUNK)], dst_v)
        pltpu.sync_copy(ext_hbm.at[pl.ds(base, CHUNK)], ext_v)

        def row_body(r, carry2):
            rows_v[r, pl.ds(0, 16)] = ext_v[r]
            return carry2

        lax.fori_loop(0, CHUNK, row_body, 0, unroll=8)
        pltpu.async_copy(rows_v, x_sh.at[dst_v], s_sem, add=True)

    def wait_s(dst_v, rows_v, s_sem):
        pltpu.make_async_copy(rows_v, x_sh.at[dst_v], s_sem).wait()

    def pair(i, carry):
        @pl.when(i > 0)
        def _():
            wait_s(dst_a, rows_a, s_a)
        do_chunk(2 * i, dst_a, ext_a, rows_a, s_a)

        @pl.when(i > 0)
        def _():
            wait_s(dst_b, rows_b, s_b)
        do_chunk(2 * i + 1, dst_b, ext_b, rows_b, s_b)
        return carry

    lax.fori_loop(0, (NCHUNK - 1) // 2, pair, 0)
    wait_s(dst_a, rows_a, s_a)
    do_chunk(NCHUNK - 1, dst_a, ext_a, rows_a, s_a)
    wait_s(dst_a, rows_a, s_a)
    wait_s(dst_b, rows_b, s_b)
    plsc.subcore_barrier()
    pltpu.sync_copy(x_sh.at[pl.ds(r0, ROWS_PER_TILE)],
                    x_out.at[c].at[pl.ds(r0, ROWS_PER_TILE)])

    @pl.when(s == 0)
    def _out_tail():
        t0 = NS * ROWS_PER_TILE
        pltpu.sync_copy(x_sh.at[pl.ds(t0, TAIL_ROWS)],
                        x_out.at[c].at[pl.ds(t0, TAIL_ROWS)])


_sc_extras = pl.kernel(
    _sc_extras_body,
    out_type=jax.ShapeDtypeStruct((NC, N, D), jnp.float32),
    mesh=_MESH,
    scratch_types=[
        pltpu.VMEM((CHUNK,), jnp.int32),
        pltpu.VMEM((CHUNK, 16), jnp.float32),
        pltpu.VMEM((CHUNK, D), jnp.float32),
        pltpu.VMEM((CHUNK,), jnp.int32),
        pltpu.VMEM((CHUNK, 16), jnp.float32),
        pltpu.VMEM((CHUNK, D), jnp.float32),
        pltpu.VMEM_SHARED((N, D), jnp.float32),
        pltpu.SemaphoreType.DMA,
        pltpu.SemaphoreType.DMA,
    ],
)


def _embed_body(an_ref, co_ref, wa_ref, ba_ref, wc_ref, bc_ref, wn_ref,
                bn_ref, h_ref):
    a = jnp.dot(an_ref[:], wa_ref[:], preferred_element_type=jnp.float32)
    a = a + ba_ref[:]
    c = jnp.dot(co_ref[:], wc_ref[:], preferred_element_type=jnp.float32)
    c = c + bc_ref[:]
    ac = jnp.concatenate([a, c], axis=1)
    h_ref[:] = jnp.dot(ac, wn_ref[:], preferred_element_type=jnp.float32) + bn_ref[:]


def _embed(atomic_num, coord, W_atom, b_atom, W_coord, b_coord, W_node, b_node):
    return pl.pallas_call(
        _embed_body,
        out_shape=jax.ShapeDtypeStruct((N, D), jnp.float32),
    )(atomic_num, coord, W_atom, b_atom.reshape(1, D),
      W_coord, b_coord.reshape(1, D), W_node, b_node.reshape(1, D))


_EB = EWP


def _edgew_body(len_ref, e0_ref, e1_ref, w0_ref, w1_ref, ext_ref):
    ln = len_ref[:]
    lg = jnp.log(ln)
    valid = lax.broadcasted_iota(jnp.int32, (_EB, 1), 0) < EW
    w0 = jnp.where(valid, jnp.exp(lg * e0_ref[0, 0]), 0.0)
    w1 = jnp.where(valid, jnp.exp(lg * e1_ref[0, 0]), 0.0)
    w0_ref[:] = jnp.broadcast_to(w0, (_EB, 16))
    w1_ref[:] = jnp.broadcast_to(w1, (_EB, 16))
    z = jnp.zeros((_EB, 12), jnp.float32)
    ext_ref[:] = jnp.concatenate([w0, w0 * ln, w1, w1 * ln, z], axis=1)


def _edgew(length, exp0, exp1):
    nb = EPAD // _EB
    return pl.pallas_call(
        _edgew_body,
        grid=(nb,),
        in_specs=[
            pl.BlockSpec((_EB, 1), lambda i: (i, 0)),
            pl.BlockSpec((1, 1), lambda i: (0, 0)),
            pl.BlockSpec((1, 1), lambda i: (0, 0)),
        ],
        out_specs=[
            pl.BlockSpec((_EB, 16), lambda i: (i, 0)),
            pl.BlockSpec((_EB, 16), lambda i: (i, 0)),
            pl.BlockSpec((_EB, 16), lambda i: (i, 0)),
        ],
        out_shape=[
            jax.ShapeDtypeStruct((EPAD, 16), jnp.float32),
            jax.ShapeDtypeStruct((EPAD, 16), jnp.float32),
            jax.ShapeDtypeStruct((EPAD, 16), jnp.float32),
        ],
    )(length, exp0.reshape(1, 1), exp1.reshape(1, 1))


_NB = 1000
_NGRID = N // _NB


def _conv_tail(s2, x2, h, agg_w, agg_b, w_edge, b_edge, wv, bv, wg, bg, eps,
               s_cols):
    S = s2[0] + s2[1]
    X = x2[0] + x2[1]
    ws = agg_w[0:D, :]
    we = agg_w[D:2 * D, :]
    wd = agg_w[2 * D:3 * D, :]
    u = jnp.dot(w_edge, we, preferred_element_type=jnp.float32)
    ce = jnp.dot(b_edge, we, preferred_element_type=jnp.float32) + agg_b
    s0 = X[:, s_cols[0]:s_cols[0] + 1]
    s1 = X[:, s_cols[1]:s_cols[1] + 1]
    agg = (jnp.dot(S, ws, preferred_element_type=jnp.float32)
           + s1 * u + s0 * ce
           + jnp.dot(s0 * h, wd, preferred_element_type=jnp.float32))
    rst = (1.0 + eps) * h + agg
    gv = jnp.dot(rst, wv, preferred_element_type=jnp.float32) + bv
    gg = jnp.dot(rst, wg, preferred_element_type=jnp.float32) + bg
    return gv * jax.nn.sigmoid(gg)


def _combine_mid_body(s2_ref, x2_ref, h_ref, aggw_ref, aggb_ref, wed_ref,
                      bed_ref, wv_ref, bv_ref, wg_ref, bg_ref, eps_ref,
                      h1_ref):
    h1_ref[:] = _conv_tail(
        s2_ref[:], x2_ref[:], h_ref[:], aggw_ref[:], aggb_ref[:], wed_ref[:],
        bed_ref[:], wv_ref[:], bv_ref[:], wg_ref[:], bg_ref[:],
        eps_ref[0, 0], (0, 1))


def _combine_mid(s2, x2, h, agg_w, agg_b, w_edge, b_edge, wv, bv, wg, bg, eps):
    full = lambda shape: pl.BlockSpec(shape, lambda i: tuple(0 for _ in shape))
    return pl.pallas_call(
        _combine_mid_body,
        grid=(_NGRID,),
        in_specs=[
            pl.BlockSpec((NC, _NB, D), lambda i: (0, i, 0)),
            pl.BlockSpec((NC, _NB, D), lambda i: (0, i, 0)),
            pl.BlockSpec((_NB, D), lambda i: (i, 0)),
            full((3 * D, D)), full((1, D)), full((1, D)), full((1, D)),
            full((D, D)), full((1, D)), full((D, D)), full((1, D)),
            full((1, 1)),
        ],
        out_specs=pl.BlockSpec((_NB, D), lambda i: (i, 0)),
        out_shape=jax.ShapeDtypeStruct((N, D), jnp.float32),
    )(s2, x2, h, agg_w, agg_b.reshape(1, D), w_edge, b_edge.reshape(1, D),
      wv, bv.reshape(1, D), wg, bg.reshape(1, D), eps.reshape(1, 1))


def _combine_last_body(s2_ref, x2_ref, h_ref, aggw_ref, aggb_ref, wed_ref,
                       bed_ref, wv_ref, bv_ref, wg_ref, bg_ref, eps_ref,
                       mask_ref, feat_ref):
    h2 = _conv_tail(
        s2_ref[:], x2_ref[:], h_ref[:], aggw_ref[:], aggb_ref[:], wed_ref[:],
        bed_ref[:], wv_ref[:], bv_ref[:], wg_ref[:], bg_ref[:],
        eps_ref[0, 0], (2, 3))
    h2 = jnp.where(mask_ref[:] == 0, 0.0, h2)
    part = jnp.sum(h2, axis=0, keepdims=True) * (1.0 / N)

    @pl.when(pl.program_id(0) == 0)
    def _init():
        feat_ref[:] = jnp.zeros_like(feat_ref)

    feat_ref[:] += part


def _combine_last(s2, x2, h, agg_w, agg_b, w_edge, b_edge, wv, bv, wg, bg,
                  eps, abs_mask):
    full = lambda shape: pl.BlockSpec(shape, lambda i: tuple(0 for _ in shape))
    return pl.pallas_call(
        _combine_last_body,
        grid=(_NGRID,),
        in_specs=[
            pl.BlockSpec((NC, _NB, D), lambda i: (0, i, 0)),
            pl.BlockSpec((NC, _NB, D), lambda i: (0, i, 0)),
            pl.BlockSpec((_NB, D), lambda i: (i, 0)),
            full((3 * D, D)), full((1, D)), full((1, D)), full((1, D)),
            full((D, D)), full((1, D)), full((D, D)), full((1, D)),
            full((1, 1)),
            pl.BlockSpec((_NB, 1), lambda i: (i, 0)),
        ],
        out_specs=pl.BlockSpec((1, D), lambda i: (0, 0)),
        out_shape=jax.ShapeDtypeStruct((1, D), jnp.float32),
    )(s2, x2, h, agg_w, agg_b.reshape(1, D), w_edge, b_edge.reshape(1, D),
      wv, bv.reshape(1, D), wg, bg.reshape(1, D), eps.reshape(1, 1),
      abs_mask.reshape(N, 1))


def _head_body(feat_ref, wm_ref, bm_ref, out_ref):
    out_ref[:] = jax.nn.sigmoid(
        jnp.dot(feat_ref[:], wm_ref[:], preferred_element_type=jnp.float32)
        + bm_ref[:])


def _head(feat, W_mlp, b_mlp):
    return pl.pallas_call(
        _head_body,
        out_shape=jax.ShapeDtypeStruct((1, OUT_DIM), jnp.float32),
    )(feat, W_mlp, b_mlp.reshape(1, OUT_DIM))


def kernel(atomic_num, coord, length, abs_mask, edge_index, W_atom, b_atom,
           W_coord, b_coord, W_node, b_node, W_edge, b_edge, agg_W0, agg_b0,
           glu_Wv0, glu_bv0, glu_Wg0, glu_bg0, exp0, eps0, agg_W1, agg_b1,
           glu_Wv1, glu_bv1, glu_Wg1, glu_bg1, exp1, eps1, W_mlp, b_mlp):
    pad2 = jnp.zeros((2, NW, EWP - EW), jnp.int32)
    pad2 = pad2 + (jnp.arange(EWP - EW, dtype=jnp.int32) % N)[None, None, :]
    ei = jnp.concatenate([edge_index.reshape(2, NW, EW), pad2], axis=2)
    ei = ei.reshape(2, EPAD)
    src = ei[0]
    dst = ei[1]
    p3 = ei.reshape(2, NW * NCHUNK, CHUNK).transpose(1, 0, 2)
    lenp = jnp.concatenate(
        [length.reshape(NW, EW), jnp.ones((NW, EWP - EW), jnp.float32)],
        axis=1).reshape(EPAD, 1)
    h0 = _embed(atomic_num, coord, W_atom, b_atom, W_coord, b_coord,
                W_node, b_node)
    w0b, w1b, wext = _edgew(lenp, exp0, exp1)
    z128 = jnp.zeros((N, D), jnp.float32)

    s_l0 = _sc_scatter(h0, p3, w0b, z128)
    x_l0 = _sc_extras(dst, wext, z128)
    h1 = _combine_mid(s_l0, x_l0, h0, agg_W0, agg_b0, W_edge, b_edge,
                      glu_Wv0, glu_bv0, glu_Wg0, glu_bg0, eps0)
    s_l1 = _sc_scatter(h1, p3, w1b, z128)
    feat = _combine_last(s_l1, x_l0, h1, agg_W1, agg_b1, W_edge, b_edge,
                         glu_Wv1, glu_bv1, glu_Wg1, glu_bg1, eps1, abs_mask)
    return _head(feat, W_mlp, b_mlp)


# speedup vs baseline: 4.0436x; 1.0001x over previous
import functools

import jax
import jax.numpy as jnp
from jax import lax
from jax.experimental import pallas as pl
from jax.experimental.pallas import tpu as pltpu
from jax.experimental.pallas import tpu_sc as plsc

N = 10000
E = 320000
D = 128
ATOM_DIM = 118
OUT_DIM = 100

NC = 2
NS = 16
NW = NC * NS
EW = E // NW
CHUNK = 80
NCHUNK = -(-EW // CHUNK)
EWP = NCHUNK * CHUNK
EPAD = NW * EWP
ROWS_PER_TILE = (N // NS) // 8 * 8
TAIL_ROWS = N - NS * ROWS_PER_TILE


_MESH = plsc.VectorSubcoreMesh(core_axis_name="c", subcore_axis_name="s")


def _sc_body(h_hbm, p3_hbm, w_hbm, z128_hbm, s_out,
             pp_a, w_a, rows_a, pp_b, w_b, rows_b,
             s_sh, g_a, g_b, s_a, s_b):
    c = lax.axis_index("c")
    s = lax.axis_index("s")
    wid = c * NS + s
    r0 = s * ROWS_PER_TILE
    pltpu.sync_copy(z128_hbm.at[pl.ds(r0, ROWS_PER_TILE)],
                    s_sh.at[pl.ds(r0, ROWS_PER_TILE)])

    @pl.when(s == 0)
    def _zero_tail():
        t0 = NS * ROWS_PER_TILE
        pltpu.sync_copy(z128_hbm.at[pl.ds(t0, TAIL_ROWS)],
                        s_sh.at[pl.ds(t0, TAIL_ROWS)])

    plsc.subcore_barrier()

    cbase = wid * NCHUNK

    def fetch(ci, pp_v, w_v, rows_v, g_sem):
        pltpu.sync_copy(p3_hbm.at[cbase + ci], pp_v)
        pltpu.sync_copy(w_hbm.at[pl.ds((cbase + ci) * CHUNK, CHUNK)], w_v)
        pltpu.async_copy(h_hbm.at[pp_v.at[0]], rows_v, g_sem)

    def scale(w_v, rows_v):
        def row_body(r, carry2):
            wb = w_v[r]
            for cb in range(D // 16):
                sl = pl.ds(cb * 16, 16)
                rows_v[r, sl] = rows_v[r, sl] * wb
            return carry2

        lax.fori_loop(0, CHUNK, row_body, 0, unroll=4)

    def wait_g(pp_v, rows_v, g_sem):
        pltpu.make_async_copy(h_hbm.at[pp_v.at[0]], rows_v, g_sem).wait()

    def wait_s(pp_v, rows_v, s_sem):
        pltpu.make_async_copy(rows_v, s_sh.at[pp_v.at[1]], s_sem).wait()

    fetch(0, pp_a, w_a, rows_a, g_a)

    def pair(i, carry):
        @pl.when(i > 0)
        def _():
            wait_s(pp_b, rows_b, s_b)
        fetch(2 * i + 1, pp_b, w_b, rows_b, g_b)
        wait_g(pp_a, rows_a, g_a)
        scale(w_a, rows_a)
        pltpu.async_copy(rows_a, s_sh.at[pp_a.at[1]], s_a, add=True)
        wait_g(pp_b, rows_b, g_b)
        scale(w_b, rows_b)
        pltpu.async_copy(rows_b, s_sh.at[pp_b.at[1]], s_b, add=True)
        wait_s(pp_a, rows_a, s_a)
        fetch(2 * i + 2, pp_a, w_a, rows_a, g_a)
        return carry

    lax.fori_loop(0, (NCHUNK - 1) // 2, pair, 0)
    wait_s(pp_b, rows_b, s_b)
    wait_g(pp_a, rows_a, g_a)
    scale(w_a, rows_a)
    pltpu.sync_copy(rows_a, s_sh.at[pp_a.at[1]], add=True)
    plsc.subcore_barrier()
    pltpu.sync_copy(s_sh.at[pl.ds(r0, ROWS_PER_TILE)],
                    s_out.at[c].at[pl.ds(r0, ROWS_PER_TILE)])

    @pl.when(s == 0)
    def _out_tail():
        t0 = NS * ROWS_PER_TILE
        pltpu.sync_copy(s_sh.at[pl.ds(t0, TAIL_ROWS)],
                        s_out.at[c].at[pl.ds(t0, TAIL_ROWS)])


_sc_scatter = pl.kernel(
    _sc_body,
    out_type=jax.ShapeDtypeStruct((NC, N, D), jnp.float32),
    mesh=_MESH,
    scratch_types=[
        pltpu.VMEM((2, CHUNK), jnp.int32),
        pltpu.VMEM((CHUNK, 16), jnp.float32),
        pltpu.VMEM((CHUNK, D), jnp.float32),
        pltpu.VMEM((2, CHUNK), jnp.int32),
        pltpu.VMEM((CHUNK, 16), jnp.float32),
        pltpu.VMEM((CHUNK, D), jnp.float32),
        pltpu.VMEM_SHARED((N, D), jnp.float32),
        pltpu.SemaphoreType.DMA,
        pltpu.SemaphoreType.DMA,
        pltpu.SemaphoreType.DMA,
        pltpu.SemaphoreType.DMA,
    ],
)


def _sc_extras_body(dst_hbm, ext_hbm, z128_hbm, x_out,
                    dst_a, ext_a, rows_a, dst_b, ext_b, rows_b, x_sh,
                    s_a, s_b):
    c = lax.axis_index("c")
    s = lax.axis_index("s")
    wid = c * NS + s
    r0 = s * ROWS_PER_TILE
    pltpu.sync_copy(z128_hbm.at[pl.ds(r0, ROWS_PER_TILE)],
                    x_sh.at[pl.ds(r0, ROWS_PER_TILE)])

    @pl.when(s == 0)
    def _zero_tail():
        t0 = NS * ROWS_PER_TILE
        pltpu.sync_copy(z128_hbm.at[pl.ds(t0, TAIL_ROWS)],
                        x_sh.at[pl.ds(t0, TAIL_ROWS)])

    zv = jnp.zeros((16,), jnp.float32)

    def zrow(r, carry):
        for cb in range(D // 16):
            rows_a[r, pl.ds(cb * 16, 16)] = zv
            rows_b[r, pl.ds(cb * 16, 16)] = zv
        return carry

    lax.fori_loop(0, CHUNK, zrow, 0, unroll=4)
    plsc.subcore_barrier()

    base0 = wid * EWP

    def do_chunk(ci, dst_v, ext_v, rows_v, s_sem):
        base = base0 + ci * CHUNK
        pltpu.sync_copy(dst_hbm.at[pl.ds(base, CHUNK)], dst_v)
        pltpu.sync_copy(ext_hbm.at[pl.ds(base, CHUNK)], ext_v)

        def row_body(r, carry2):
            rows_v[r, pl.ds(0, 16)] = ext_v[r]
            return carry2

        lax.fori_loop(0, CHUNK, row_body, 0, unroll=8)
        pltpu.async_copy(rows_v, x_sh.at[dst_v], s_sem, add=True)

    def wait_s(dst_v, rows_v, s_sem):
        pltpu.make_async_copy(rows_v, x_sh.at[dst_v], s_sem).wait()

    def pair(i, carry):
        @pl.when(i > 0)
        def _():
            wait_s(dst_a, rows_a, s_a)
        do_chunk(2 * i, dst_a, ext_a, rows_a, s_a)

        @pl.when(i > 0)
        def _():
            wait_s(dst_b, rows_b, s_b)
        do_chunk(2 * i + 1, dst_b, ext_b, rows_b, s_b)
        return carry

    lax.fori_loop(0, (NCHUNK - 1) // 2, pair, 0)
    wait_s(dst_a, rows_a, s_a)
    do_chunk(NCHUNK - 1, dst_a, ext_a, rows_a, s_a)
    wait_s(dst_a, rows_a, s_a)
    wait_s(dst_b, rows_b, s_b)
    plsc.subcore_barrier()
    pltpu.sync_copy(x_sh.at[pl.ds(r0, ROWS_PER_TILE)],
                    x_out.at[c].at[pl.ds(r0, ROWS_PER_TILE)])

    @pl.when(s == 0)
    def _out_tail():
        t0 = NS * ROWS_PER_TILE
        pltpu.sync_copy(x_sh.at[pl.ds(t0, TAIL_ROWS)],
                        x_out.at[c].at[pl.ds(t0, TAIL_ROWS)])


_sc_extras = pl.kernel(
    _sc_extras_body,
    out_type=jax.ShapeDtypeStruct((NC, N, D), jnp.float32),
    mesh=_MESH,
    scratch_types=[
        pltpu.VMEM((CHUNK,), jnp.int32),
        pltpu.VMEM((CHUNK, 16), jnp.float32),
        pltpu.VMEM((CHUNK, D), jnp.float32),
        pltpu.VMEM((CHUNK,), jnp.int32),
        pltpu.VMEM((CHUNK, 16), jnp.float32),
        pltpu.VMEM((CHUNK, D), jnp.float32),
        pltpu.VMEM_SHARED((N, D), jnp.float32),
        pltpu.SemaphoreType.DMA,
        pltpu.SemaphoreType.DMA,
    ],
)


def _embed_body(an_ref, co_ref, wa_ref, ba_ref, wc_ref, bc_ref, wn_ref,
                bn_ref, h_ref):
    a = jnp.dot(an_ref[:], wa_ref[:], preferred_element_type=jnp.float32)
    a = a + ba_ref[:]
    c = jnp.dot(co_ref[:], wc_ref[:], preferred_element_type=jnp.float32)
    c = c + bc_ref[:]
    ac = jnp.concatenate([a, c], axis=1)
    h_ref[:] = jnp.dot(ac, wn_ref[:], preferred_element_type=jnp.float32) + bn_ref[:]


def _embed(atomic_num, coord, W_atom, b_atom, W_coord, b_coord, W_node, b_node):
    return pl.pallas_call(
        _embed_body,
        out_shape=jax.ShapeDtypeStruct((N, D), jnp.float32),
    )(atomic_num, coord, W_atom, b_atom.reshape(1, D),
      W_coord, b_coord.reshape(1, D), W_node, b_node.reshape(1, D))


_EB = EWP


def _edgew_body(len_ref, e0_ref, e1_ref, w0_ref, w1_ref, ext_ref):
    ln = len_ref[:]
    lg = jnp.log(ln)
    valid = lax.broadcasted_iota(jnp.int32, (_EB, 1), 0) < EW
    w0 = jnp.where(valid, jnp.exp(lg * e0_ref[0, 0]), 0.0)
    w1 = jnp.where(valid, jnp.exp(lg * e1_ref[0, 0]), 0.0)
    w0_ref[:] = jnp.broadcast_to(w0, (_EB, 16))
    w1_ref[:] = jnp.broadcast_to(w1, (_EB, 16))
    z = jnp.zeros((_EB, 12), jnp.float32)
    ext_ref[:] = jnp.concatenate([w0, w0 * ln, w1, w1 * ln, z], axis=1)


def _edgew(length, exp0, exp1):
    nb = EPAD // _EB
    return pl.pallas_call(
        _edgew_body,
        grid=(nb,),
        in_specs=[
            pl.BlockSpec((_EB, 1), lambda i: (i, 0)),
            pl.BlockSpec((1, 1), lambda i: (0, 0)),
            pl.BlockSpec((1, 1), lambda i: (0, 0)),
        ],
        out_specs=[
            pl.BlockSpec((_EB, 16), lambda i: (i, 0)),
            pl.BlockSpec((_EB, 16), lambda i: (i, 0)),
            pl.BlockSpec((_EB, 16), lambda i: (i, 0)),
        ],
        out_shape=[
            jax.ShapeDtypeStruct((EPAD, 16), jnp.float32),
            jax.ShapeDtypeStruct((EPAD, 16), jnp.float32),
            jax.ShapeDtypeStruct((EPAD, 16), jnp.float32),
        ],
    )(length, exp0.reshape(1, 1), exp1.reshape(1, 1))


_NB = 1000
_NGRID = N // _NB


def _conv_tail(s2, x2, h, agg_w, agg_b, w_edge, b_edge, wv, bv, wg, bg, eps,
               s_cols):
    S = s2[0] + s2[1]
    X = x2[0] + x2[1]
    ws = agg_w[0:D, :]
    we = agg_w[D:2 * D, :]
    wd = agg_w[2 * D:3 * D, :]
    u = jnp.dot(w_edge, we, preferred_element_type=jnp.float32)
    ce = jnp.dot(b_edge, we, preferred_element_type=jnp.float32) + agg_b
    s0 = X[:, s_cols[0]:s_cols[0] + 1]
    s1 = X[:, s_cols[1]:s_cols[1] + 1]
    agg = (jnp.dot(S, ws, preferred_element_type=jnp.float32)
           + s1 * u + s0 * ce
           + jnp.dot(s0 * h, wd, preferred_element_type=jnp.float32))
    rst = (1.0 + eps) * h + agg
    gv = jnp.dot(rst, wv, preferred_element_type=jnp.float32) + bv
    gg = jnp.dot(rst, wg, preferred_element_type=jnp.float32) + bg
    return gv * jax.nn.sigmoid(gg)


def _combine_mid_body(s2_ref, x2_ref, h_ref, aggw_ref, aggb_ref, wed_ref,
                      bed_ref, wv_ref, bv_ref, wg_ref, bg_ref, eps_ref,
                      h1_ref):
    h1_ref[:] = _conv_tail(
        s2_ref[:], x2_ref[:], h_ref[:], aggw_ref[:], aggb_ref[:], wed_ref[:],
        bed_ref[:], wv_ref[:], bv_ref[:], wg_ref[:], bg_ref[:],
        eps_ref[0, 0], (0, 1))


def _combine_mid(s2, x2, h, agg_w, agg_b, w_edge, b_edge, wv, bv, wg, bg, eps):
    full = lambda shape: pl.BlockSpec(shape, lambda i: tuple(0 for _ in shape))
    return pl.pallas_call(
        _combine_mid_body,
        grid=(_NGRID,),
        in_specs=[
            pl.BlockSpec((NC, _NB, D), lambda i: (0, i, 0)),
            pl.BlockSpec((NC, _NB, 8), lambda i: (0, i, 0)),
            pl.BlockSpec((_NB, D), lambda i: (i, 0)),
            full((3 * D, D)), full((1, D)), full((1, D)), full((1, D)),
            full((D, D)), full((1, D)), full((D, D)), full((1, D)),
            full((1, 1)),
        ],
        out_specs=pl.BlockSpec((_NB, D), lambda i: (i, 0)),
        out_shape=jax.ShapeDtypeStruct((N, D), jnp.float32),
    )(s2, x2, h, agg_w, agg_b.reshape(1, D), w_edge, b_edge.reshape(1, D),
      wv, bv.reshape(1, D), wg, bg.reshape(1, D), eps.reshape(1, 1))


def _combine_last_body(s2_ref, x2_ref, h_ref, aggw_ref, aggb_ref, wed_ref,
                       bed_ref, wv_ref, bv_ref, wg_ref, bg_ref, eps_ref,
                       mask_ref, wm_ref, bm_ref, out_ref, feat_ref):
    h2 = _conv_tail(
        s2_ref[:], x2_ref[:], h_ref[:], aggw_ref[:], aggb_ref[:], wed_ref[:],
        bed_ref[:], wv_ref[:], bv_ref[:], wg_ref[:], bg_ref[:],
        eps_ref[0, 0], (2, 3))
    h2 = jnp.where(mask_ref[:] == 0, 0.0, h2)
    part = jnp.sum(h2, axis=0, keepdims=True) * (1.0 / N)

    @pl.when(pl.program_id(0) == 0)
    def _init():
        feat_ref[:] = jnp.zeros_like(feat_ref)

    feat_ref[:] += part

    @pl.when(pl.program_id(0) == _NGRID - 1)
    def _head():
        out_ref[:] = jax.nn.sigmoid(
            jnp.dot(feat_ref[:], wm_ref[:],
                    preferred_element_type=jnp.float32) + bm_ref[:])


def _combine_last(s2, x2, h, agg_w, agg_b, w_edge, b_edge, wv, bv, wg, bg,
                  eps, abs_mask, w_mlp, b_mlp):
    full = lambda shape: pl.BlockSpec(shape, lambda i: tuple(0 for _ in shape))
    out, _ = pl.pallas_call(
        _combine_last_body,
        grid=(_NGRID,),
        in_specs=[
            pl.BlockSpec((NC, _NB, D), lambda i: (0, i, 0)),
            pl.BlockSpec((NC, _NB, 8), lambda i: (0, i, 0)),
            pl.BlockSpec((_NB, D), lambda i: (i, 0)),
            full((3 * D, D)), full((1, D)), full((1, D)), full((1, D)),
            full((D, D)), full((1, D)), full((D, D)), full((1, D)),
            full((1, 1)),
            pl.BlockSpec((_NB, 1), lambda i: (i, 0)),
            full((D, OUT_DIM)), full((1, OUT_DIM)),
        ],
        out_specs=[pl.BlockSpec((1, OUT_DIM), lambda i: (0, 0)),
                   pl.BlockSpec((1, D), lambda i: (0, 0))],
        out_shape=[jax.ShapeDtypeStruct((1, OUT_DIM), jnp.float32),
                   jax.ShapeDtypeStruct((1, D), jnp.float32)],
    )(s2, x2, h, agg_w, agg_b.reshape(1, D), w_edge, b_edge.reshape(1, D),
      wv, bv.reshape(1, D), wg, bg.reshape(1, D), eps.reshape(1, 1),
      abs_mask.reshape(N, 1), w_mlp, b_mlp.reshape(1, OUT_DIM))
    return out


def _head_body(feat_ref, wm_ref, bm_ref, out_ref):
    out_ref[:] = jax.nn.sigmoid(
        jnp.dot(feat_ref[:], wm_ref[:], preferred_element_type=jnp.float32)
        + bm_ref[:])


def _head(feat, W_mlp, b_mlp):
    return pl.pallas_call(
        _head_body,
        out_shape=jax.ShapeDtypeStruct((1, OUT_DIM), jnp.float32),
    )(feat, W_mlp, b_mlp.reshape(1, OUT_DIM))


def kernel(atomic_num, coord, length, abs_mask, edge_index, W_atom, b_atom,
           W_coord, b_coord, W_node, b_node, W_edge, b_edge, agg_W0, agg_b0,
           glu_Wv0, glu_bv0, glu_Wg0, glu_bg0, exp0, eps0, agg_W1, agg_b1,
           glu_Wv1, glu_bv1, glu_Wg1, glu_bg1, exp1, eps1, W_mlp, b_mlp):
    pad2 = jnp.zeros((2, NW, EWP - EW), jnp.int32)
    pad2 = pad2 + (jnp.arange(EWP - EW, dtype=jnp.int32) % N)[None, None, :]
    ei = jnp.concatenate([edge_index.reshape(2, NW, EW), pad2], axis=2)
    ei = ei.reshape(2, EPAD)
    src = ei[0]
    dst = ei[1]
    p3 = ei.reshape(2, NW * NCHUNK, CHUNK).transpose(1, 0, 2)
    lenp = jnp.concatenate(
        [length.reshape(NW, EW), jnp.ones((NW, EWP - EW), jnp.float32)],
        axis=1).reshape(EPAD, 1)
    h0 = _embed(atomic_num, coord, W_atom, b_atom, W_coord, b_coord,
                W_node, b_node)
    w0b, w1b, wext = _edgew(lenp, exp0, exp1)
    z128 = jnp.zeros((N, D), jnp.float32)

    s_l0 = _sc_scatter(h0, p3, w0b, z128)
    x_l0 = _sc_extras(dst, wext, z128)
    x8 = x_l0[:, :, :8]
    h1 = _combine_mid(s_l0, x8, h0, agg_W0, agg_b0, W_edge, b_edge,
                      glu_Wv0, glu_bv0, glu_Wg0, glu_bg0, eps0)
    s_l1 = _sc_scatter(h1, p3, w1b, z128)
    return _combine_last(s_l1, x8, h1, agg_W1, agg_b1, W_edge, b_edge,
                         glu_Wv1, glu_bv1, glu_Wg1, glu_bg1, eps1, abs_mask,
                         W_mlp, b_mlp)


# speedup vs baseline: 4.3126x; 1.0665x over previous
import functools

import jax
import jax.numpy as jnp
from jax import lax
from jax.experimental import pallas as pl
from jax.experimental.pallas import tpu as pltpu
from jax.experimental.pallas import tpu_sc as plsc

N = 10000
E = 320000
D = 128
ATOM_DIM = 118
OUT_DIM = 100

NC = 2
NS = 16
NW = NC * NS
EW = E // NW
CHUNK = 80
NCHUNK = -(-EW // CHUNK)
EWP = NCHUNK * CHUNK
EPAD = NW * EWP
ROWS_PER_TILE = (N // NS) // 8 * 8
TAIL_ROWS = N - NS * ROWS_PER_TILE


_MESH = plsc.VectorSubcoreMesh(core_axis_name="c", subcore_axis_name="s")


def _sc_body(h_hbm, p3_hbm, w_hbm, z128_hbm, s_out,
             pp_a, w_a, rows_a, pp_b, w_b, rows_b,
             s_sh, g_a, g_b, s_a, s_b, i_a, i_b):
    c = lax.axis_index("c")
    s = lax.axis_index("s")
    wid = c * NS + s
    r0 = s * ROWS_PER_TILE
    pltpu.sync_copy(z128_hbm.at[pl.ds(r0, ROWS_PER_TILE)],
                    s_sh.at[pl.ds(r0, ROWS_PER_TILE)])

    @pl.when(s == 0)
    def _zero_tail():
        t0 = NS * ROWS_PER_TILE
        pltpu.sync_copy(z128_hbm.at[pl.ds(t0, TAIL_ROWS)],
                        s_sh.at[pl.ds(t0, TAIL_ROWS)])

    plsc.subcore_barrier()

    cbase = wid * NCHUNK

    def fetch_idx(ci, pp_v, w_v, i_sem):
        pltpu.async_copy(p3_hbm.at[cbase + ci], pp_v, i_sem)
        pltpu.async_copy(w_hbm.at[pl.ds((cbase + ci) * CHUNK, CHUNK)], w_v,
                         i_sem)

    def gather(ci, pp_v, w_v, rows_v, i_sem, g_sem):
        pltpu.make_async_copy(p3_hbm.at[cbase + ci], pp_v, i_sem).wait()
        pltpu.make_async_copy(
            w_hbm.at[pl.ds((cbase + ci) * CHUNK, CHUNK)], w_v, i_sem).wait()
        pltpu.async_copy(h_hbm.at[pp_v.at[0]], rows_v, g_sem)

    def scale(w_v, rows_v):
        def row_body(r, carry2):
            wb = w_v[r]
            for cb in range(D // 16):
                sl = pl.ds(cb * 16, 16)
                rows_v[r, sl] = rows_v[r, sl] * wb
            return carry2

        lax.fori_loop(0, CHUNK, row_body, 0, unroll=4)

    def wait_g(pp_v, rows_v, g_sem):
        pltpu.make_async_copy(h_hbm.at[pp_v.at[0]], rows_v, g_sem).wait()

    def wait_s(pp_v, rows_v, s_sem):
        pltpu.make_async_copy(rows_v, s_sh.at[pp_v.at[1]], s_sem).wait()

    fetch_idx(0, pp_a, w_a, i_a)
    gather(0, pp_a, w_a, rows_a, i_a, g_a)

    def pair(i, carry):
        @pl.when(i > 0)
        def _():
            wait_s(pp_b, rows_b, s_b)
        fetch_idx(2 * i + 1, pp_b, w_b, i_b)
        gather(2 * i + 1, pp_b, w_b, rows_b, i_b, g_b)
        wait_g(pp_a, rows_a, g_a)
        scale(w_a, rows_a)
        pltpu.async_copy(rows_a, s_sh.at[pp_a.at[1]], s_a, add=True)
        wait_g(pp_b, rows_b, g_b)
        scale(w_b, rows_b)
        pltpu.async_copy(rows_b, s_sh.at[pp_b.at[1]], s_b, add=True)
        wait_s(pp_a, rows_a, s_a)
        fetch_idx(2 * i + 2, pp_a, w_a, i_a)
        gather(2 * i + 2, pp_a, w_a, rows_a, i_a, g_a)
        return carry

    lax.fori_loop(0, (NCHUNK - 1) // 2, pair, 0)
    wait_s(pp_b, rows_b, s_b)
    wait_g(pp_a, rows_a, g_a)
    scale(w_a, rows_a)
    pltpu.sync_copy(rows_a, s_sh.at[pp_a.at[1]], add=True)
    plsc.subcore_barrier()
    pltpu.sync_copy(s_sh.at[pl.ds(r0, ROWS_PER_TILE)],
                    s_out.at[c].at[pl.ds(r0, ROWS_PER_TILE)])

    @pl.when(s == 0)
    def _out_tail():
        t0 = NS * ROWS_PER_TILE
        pltpu.sync_copy(s_sh.at[pl.ds(t0, TAIL_ROWS)],
                        s_out.at[c].at[pl.ds(t0, TAIL_ROWS)])


_sc_scatter = pl.kernel(
    _sc_body,
    out_type=jax.ShapeDtypeStruct((NC, N, D), jnp.float32),
    mesh=_MESH,
    scratch_types=[
        pltpu.VMEM((2, CHUNK), jnp.int32),
        pltpu.VMEM((CHUNK, 16), jnp.float32),
        pltpu.VMEM((CHUNK, D), jnp.float32),
        pltpu.VMEM((2, CHUNK), jnp.int32),
        pltpu.VMEM((CHUNK, 16), jnp.float32),
        pltpu.VMEM((CHUNK, D), jnp.float32),
        pltpu.VMEM_SHARED((N, D), jnp.float32),
        pltpu.SemaphoreType.DMA,
        pltpu.SemaphoreType.DMA,
        pltpu.SemaphoreType.DMA,
        pltpu.SemaphoreType.DMA,
        pltpu.SemaphoreType.DMA,
        pltpu.SemaphoreType.DMA,
    ],
)


def _sc_extras_body(dst_hbm, ext_hbm, z128_hbm, x_out,
                    dst_a, ext_a, rows_a, dst_b, ext_b, rows_b, x_sh,
                    s_a, s_b):
    c = lax.axis_index("c")
    s = lax.axis_index("s")
    wid = c * NS + s
    r0 = s * ROWS_PER_TILE
    pltpu.sync_copy(z128_hbm.at[pl.ds(r0, ROWS_PER_TILE)],
                    x_sh.at[pl.ds(r0, ROWS_PER_TILE)])

    @pl.when(s == 0)
    def _zero_tail():
        t0 = NS * ROWS_PER_TILE
        pltpu.sync_copy(z128_hbm.at[pl.ds(t0, TAIL_ROWS)],
                        x_sh.at[pl.ds(t0, TAIL_ROWS)])

    zv = jnp.zeros((16,), jnp.float32)

    def zrow(r, carry):
        for cb in range(D // 16):
            rows_a[r, pl.ds(cb * 16, 16)] = zv
            rows_b[r, pl.ds(cb * 16, 16)] = zv
        return carry

    lax.fori_loop(0, CHUNK, zrow, 0, unroll=4)
    plsc.subcore_barrier()

    base0 = wid * EWP

    def do_chunk(ci, dst_v, ext_v, rows_v, s_sem):
        base = base0 + ci * CHUNK
        pltpu.sync_copy(dst_hbm.at[pl.ds(base, CHUNK)], dst_v)
        pltpu.sync_copy(ext_hbm.at[pl.ds(base, CHUNK)], ext_v)

        def row_body(r, carry2):
            rows_v[r, pl.ds(0, 16)] = ext_v[r]
            return carry2

        lax.fori_loop(0, CHUNK, row_body, 0, unroll=8)
        pltpu.async_copy(rows_v, x_sh.at[dst_v], s_sem, add=True)

    def wait_s(dst_v, rows_v, s_sem):
        pltpu.make_async_copy(rows_v, x_sh.at[dst_v], s_sem).wait()

    def pair(i, carry):
        @pl.when(i > 0)
        def _():
            wait_s(dst_a, rows_a, s_a)
        do_chunk(2 * i, dst_a, ext_a, rows_a, s_a)

        @pl.when(i > 0)
        def _():
            wait_s(dst_b, rows_b, s_b)
        do_chunk(2 * i + 1, dst_b, ext_b, rows_b, s_b)
        return carry

    lax.fori_loop(0, (NCHUNK - 1) // 2, pair, 0)
    wait_s(dst_a, rows_a, s_a)
    do_chunk(NCHUNK - 1, dst_a, ext_a, rows_a, s_a)
    wait_s(dst_a, rows_a, s_a)
    wait_s(dst_b, rows_b, s_b)
    plsc.subcore_barrier()
    pltpu.sync_copy(x_sh.at[pl.ds(r0, ROWS_PER_TILE)],
                    x_out.at[c].at[pl.ds(r0, ROWS_PER_TILE)])

    @pl.when(s == 0)
    def _out_tail():
        t0 = NS * ROWS_PER_TILE
        pltpu.sync_copy(x_sh.at[pl.ds(t0, TAIL_ROWS)],
                        x_out.at[c].at[pl.ds(t0, TAIL_ROWS)])


_sc_extras = pl.kernel(
    _sc_extras_body,
    out_type=jax.ShapeDtypeStruct((NC, N, D), jnp.float32),
    mesh=_MESH,
    scratch_types=[
        pltpu.VMEM((CHUNK,), jnp.int32),
        pltpu.VMEM((CHUNK, 16), jnp.float32),
        pltpu.VMEM((CHUNK, D), jnp.float32),
        pltpu.VMEM((CHUNK,), jnp.int32),
        pltpu.VMEM((CHUNK, 16), jnp.float32),
        pltpu.VMEM((CHUNK, D), jnp.float32),
        pltpu.VMEM_SHARED((N, D), jnp.float32),
        pltpu.SemaphoreType.DMA,
        pltpu.SemaphoreType.DMA,
    ],
)


def _embed_body(an_ref, co_ref, wa_ref, ba_ref, wc_ref, bc_ref, wn_ref,
                bn_ref, h_ref):
    a = jnp.dot(an_ref[:], wa_ref[:], preferred_element_type=jnp.float32)
    a = a + ba_ref[:]
    c = jnp.dot(co_ref[:], wc_ref[:], preferred_element_type=jnp.float32)
    c = c + bc_ref[:]
    ac = jnp.concatenate([a, c], axis=1)
    h_ref[:] = jnp.dot(ac, wn_ref[:], preferred_element_type=jnp.float32) + bn_ref[:]


def _embed(atomic_num, coord, W_atom, b_atom, W_coord, b_coord, W_node, b_node):
    return pl.pallas_call(
        _embed_body,
        out_shape=jax.ShapeDtypeStruct((N, D), jnp.float32),
    )(atomic_num, coord, W_atom, b_atom.reshape(1, D),
      W_coord, b_coord.reshape(1, D), W_node, b_node.reshape(1, D))


_EB = EWP


def _edgew_body(len_ref, e0_ref, e1_ref, w0_ref, w1_ref, ext_ref):
    ln = len_ref[:]
    lg = jnp.log(ln)
    valid = lax.broadcasted_iota(jnp.int32, (_EB, 1), 0) < EW
    w0 = jnp.where(valid, jnp.exp(lg * e0_ref[0, 0]), 0.0)
    w1 = jnp.where(valid, jnp.exp(lg * e1_ref[0, 0]), 0.0)
    w0_ref[:] = jnp.broadcast_to(w0, (_EB, 16))
    w1_ref[:] = jnp.broadcast_to(w1, (_EB, 16))
    z = jnp.zeros((_EB, 12), jnp.float32)
    ext_ref[:] = jnp.concatenate([w0, w0 * ln, w1, w1 * ln, z], axis=1)


def _edgew(length, exp0, exp1):
    nb = EPAD // _EB
    return pl.pallas_call(
        _edgew_body,
        grid=(nb,),
        in_specs=[
            pl.BlockSpec((_EB, 1), lambda i: (i, 0)),
            pl.BlockSpec((1, 1), lambda i: (0, 0)),
            pl.BlockSpec((1, 1), lambda i: (0, 0)),
        ],
        out_specs=[
            pl.BlockSpec((_EB, 16), lambda i: (i, 0)),
            pl.BlockSpec((_EB, 16), lambda i: (i, 0)),
            pl.BlockSpec((_EB, 16), lambda i: (i, 0)),
        ],
        out_shape=[
            jax.ShapeDtypeStruct((EPAD, 16), jnp.float32),
            jax.ShapeDtypeStruct((EPAD, 16), jnp.float32),
            jax.ShapeDtypeStruct((EPAD, 16), jnp.float32),
        ],
    )(length, exp0.reshape(1, 1), exp1.reshape(1, 1))


_NB = 1000
_NGRID = N // _NB


def _conv_tail(s2, x2, h, agg_w, agg_b, w_edge, b_edge, wv, bv, wg, bg, eps,
               s_cols):
    S = s2[0] + s2[1]
    X = x2[0] + x2[1]
    ws = agg_w[0:D, :]
    we = agg_w[D:2 * D, :]
    wd = agg_w[2 * D:3 * D, :]
    u = jnp.dot(w_edge, we, preferred_element_type=jnp.float32)
    ce = jnp.dot(b_edge, we, preferred_element_type=jnp.float32) + agg_b
    s0 = X[:, s_cols[0]:s_cols[0] + 1]
    s1 = X[:, s_cols[1]:s_cols[1] + 1]
    agg = (jnp.dot(S, ws, preferred_element_type=jnp.float32)
           + s1 * u + s0 * ce
           + jnp.dot(s0 * h, wd, preferred_element_type=jnp.float32))
    rst = (1.0 + eps) * h + agg
    gv = jnp.dot(rst, wv, preferred_element_type=jnp.float32) + bv
    gg = jnp.dot(rst, wg, preferred_element_type=jnp.float32) + bg
    return gv * jax.nn.sigmoid(gg)


def _combine_mid_body(s2_ref, x2_ref, h_ref, aggw_ref, aggb_ref, wed_ref,
                      bed_ref, wv_ref, bv_ref, wg_ref, bg_ref, eps_ref,
                      h1_ref):
    h1_ref[:] = _conv_tail(
        s2_ref[:], x2_ref[:], h_ref[:], aggw_ref[:], aggb_ref[:], wed_ref[:],
        bed_ref[:], wv_ref[:], bv_ref[:], wg_ref[:], bg_ref[:],
        eps_ref[0, 0], (0, 1))


def _combine_mid(s2, x2, h, agg_w, agg_b, w_edge, b_edge, wv, bv, wg, bg, eps):
    full = lambda shape: pl.BlockSpec(shape, lambda i: tuple(0 for _ in shape))
    return pl.pallas_call(
        _combine_mid_body,
        grid=(_NGRID,),
        in_specs=[
            pl.BlockSpec((NC, _NB, D), lambda i: (0, i, 0)),
            pl.BlockSpec((NC, _NB, 8), lambda i: (0, i, 0)),
            pl.BlockSpec((_NB, D), lambda i: (i, 0)),
            full((3 * D, D)), full((1, D)), full((1, D)), full((1, D)),
            full((D, D)), full((1, D)), full((D, D)), full((1, D)),
            full((1, 1)),
        ],
        out_specs=pl.BlockSpec((_NB, D), lambda i: (i, 0)),
        out_shape=jax.ShapeDtypeStruct((N, D), jnp.float32),
    )(s2, x2, h, agg_w, agg_b.reshape(1, D), w_edge, b_edge.reshape(1, D),
      wv, bv.reshape(1, D), wg, bg.reshape(1, D), eps.reshape(1, 1))


def _combine_last_body(s2_ref, x2_ref, h_ref, aggw_ref, aggb_ref, wed_ref,
                       bed_ref, wv_ref, bv_ref, wg_ref, bg_ref, eps_ref,
                       mask_ref, wm_ref, bm_ref, out_ref, feat_ref):
    h2 = _conv_tail(
        s2_ref[:], x2_ref[:], h_ref[:], aggw_ref[:], aggb_ref[:], wed_ref[:],
        bed_ref[:], wv_ref[:], bv_ref[:], wg_ref[:], bg_ref[:],
        eps_ref[0, 0], (2, 3))
    h2 = jnp.where(mask_ref[:] == 0, 0.0, h2)
    part = jnp.sum(h2, axis=0, keepdims=True) * (1.0 / N)

    @pl.when(pl.program_id(0) == 0)
    def _init():
        feat_ref[:] = jnp.zeros_like(feat_ref)

    feat_ref[:] += part

    @pl.when(pl.program_id(0) == _NGRID - 1)
    def _head():
        out_ref[:] = jax.nn.sigmoid(
            jnp.dot(feat_ref[:], wm_ref[:],
                    preferred_element_type=jnp.float32) + bm_ref[:])


def _combine_last(s2, x2, h, agg_w, agg_b, w_edge, b_edge, wv, bv, wg, bg,
                  eps, abs_mask, w_mlp, b_mlp):
    full = lambda shape: pl.BlockSpec(shape, lambda i: tuple(0 for _ in shape))
    out, _ = pl.pallas_call(
        _combine_last_body,
        grid=(_NGRID,),
        in_specs=[
            pl.BlockSpec((NC, _NB, D), lambda i: (0, i, 0)),
            pl.BlockSpec((NC, _NB, 8), lambda i: (0, i, 0)),
            pl.BlockSpec((_NB, D), lambda i: (i, 0)),
            full((3 * D, D)), full((1, D)), full((1, D)), full((1, D)),
            full((D, D)), full((1, D)), full((D, D)), full((1, D)),
            full((1, 1)),
            pl.BlockSpec((_NB, 1), lambda i: (i, 0)),
            full((D, OUT_DIM)), full((1, OUT_DIM)),
        ],
        out_specs=[pl.BlockSpec((1, OUT_DIM), lambda i: (0, 0)),
                   pl.BlockSpec((1, D), lambda i: (0, 0))],
        out_shape=[jax.ShapeDtypeStruct((1, OUT_DIM), jnp.float32),
                   jax.ShapeDtypeStruct((1, D), jnp.float32)],
    )(s2, x2, h, agg_w, agg_b.reshape(1, D), w_edge, b_edge.reshape(1, D),
      wv, bv.reshape(1, D), wg, bg.reshape(1, D), eps.reshape(1, 1),
      abs_mask.reshape(N, 1), w_mlp, b_mlp.reshape(1, OUT_DIM))
    return out


def _head_body(feat_ref, wm_ref, bm_ref, out_ref):
    out_ref[:] = jax.nn.sigmoid(
        jnp.dot(feat_ref[:], wm_ref[:], preferred_element_type=jnp.float32)
        + bm_ref[:])


def _head(feat, W_mlp, b_mlp):
    return pl.pallas_call(
        _head_body,
        out_shape=jax.ShapeDtypeStruct((1, OUT_DIM), jnp.float32),
    )(feat, W_mlp, b_mlp.reshape(1, OUT_DIM))


def kernel(atomic_num, coord, length, abs_mask, edge_index, W_atom, b_atom,
           W_coord, b_coord, W_node, b_node, W_edge, b_edge, agg_W0, agg_b0,
           glu_Wv0, glu_bv0, glu_Wg0, glu_bg0, exp0, eps0, agg_W1, agg_b1,
           glu_Wv1, glu_bv1, glu_Wg1, glu_bg1, exp1, eps1, W_mlp, b_mlp):
    pad2 = jnp.zeros((2, NW, EWP - EW), jnp.int32)
    pad2 = pad2 + (jnp.arange(EWP - EW, dtype=jnp.int32) % N)[None, None, :]
    ei = jnp.concatenate([edge_index.reshape(2, NW, EW), pad2], axis=2)
    ei = ei.reshape(2, EPAD)
    src = ei[0]
    dst = ei[1]
    p3 = ei.reshape(2, NW * NCHUNK, CHUNK).transpose(1, 0, 2)
    lenp = jnp.concatenate(
        [length.reshape(NW, EW), jnp.ones((NW, EWP - EW), jnp.float32)],
        axis=1).reshape(EPAD, 1)
    h0 = _embed(atomic_num, coord, W_atom, b_atom, W_coord, b_coord,
                W_node, b_node)
    w0b, w1b, wext = _edgew(lenp, exp0, exp1)
    z128 = jnp.zeros((N, D), jnp.float32)

    s_l0 = _sc_scatter(h0, p3, w0b, z128)
    x_l0 = _sc_extras(dst, wext, z128)
    x8 = x_l0[:, :, :8]
    h1 = _combine_mid(s_l0, x8, h0, agg_W0, agg_b0, W_edge, b_edge,
                      glu_Wv0, glu_bv0, glu_Wg0, glu_bg0, eps0)
    s_l1 = _sc_scatter(h1, p3, w1b, z128)
    return _combine_last(s_l1, x8, h1, agg_W1, agg_b1, W_edge, b_edge,
                         glu_Wv1, glu_bv1, glu_Wg1, glu_bg1, eps1, abs_mask,
                         W_mlp, b_mlp)


# speedup vs baseline: 4.3148x; 1.0005x over previous
import functools

import jax
import jax.numpy as jnp
from jax import lax
from jax.experimental import pallas as pl
from jax.experimental.pallas import tpu as pltpu
from jax.experimental.pallas import tpu_sc as plsc

N = 10000
E = 320000
D = 128
ATOM_DIM = 118
OUT_DIM = 100

NC = 2
NS = 16
NW = NC * NS
EW = E // NW
CHUNK = 80
NCHUNK = -(-EW // CHUNK)
EWP = NCHUNK * CHUNK
EPAD = NW * EWP
ROWS_PER_TILE = (N // NS) // 8 * 8
TAIL_ROWS = N - NS * ROWS_PER_TILE


_MESH = plsc.VectorSubcoreMesh(core_axis_name="c", subcore_axis_name="s")


def _sc_body(h_hbm, p3_hbm, w_hbm, z128_hbm, s_out,
             pp_a, w_a, rows_a, pp_b, w_b, rows_b,
             s_sh, g_a, g_b, s_a, s_b, i_a, i_b):
    c = lax.axis_index("c")
    s = lax.axis_index("s")
    wid = c * NS + s
    r0 = s * ROWS_PER_TILE
    pltpu.sync_copy(z128_hbm.at[pl.ds(r0, ROWS_PER_TILE)],
                    s_sh.at[pl.ds(r0, ROWS_PER_TILE)])

    @pl.when(s == 0)
    def _zero_tail():
        t0 = NS * ROWS_PER_TILE
        pltpu.sync_copy(z128_hbm.at[pl.ds(t0, TAIL_ROWS)],
                        s_sh.at[pl.ds(t0, TAIL_ROWS)])

    plsc.subcore_barrier()

    cbase = wid * NCHUNK

    def fetch_idx(ci, pp_v, w_v, i_sem):
        pltpu.async_copy(p3_hbm.at[cbase + ci], pp_v, i_sem)
        pltpu.async_copy(w_hbm.at[pl.ds((cbase + ci) * CHUNK, CHUNK)], w_v,
                         i_sem)

    def gather(ci, pp_v, w_v, rows_v, i_sem, g_sem):
        pltpu.make_async_copy(p3_hbm.at[cbase + ci], pp_v, i_sem).wait()
        pltpu.make_async_copy(
            w_hbm.at[pl.ds((cbase + ci) * CHUNK, CHUNK)], w_v, i_sem).wait()
        pltpu.async_copy(h_hbm.at[pp_v.at[0]], rows_v, g_sem)

    def scale(w_v, rows_v):
        def row_body(r, carry2):
            wb = w_v[r]
            for cb in range(D // 16):
                sl = pl.ds(cb * 16, 16)
                rows_v[r, sl] = rows_v[r, sl] * wb
            return carry2

        lax.fori_loop(0, CHUNK, row_body, 0, unroll=4)

    def wait_g(pp_v, rows_v, g_sem):
        pltpu.make_async_copy(h_hbm.at[pp_v.at[0]], rows_v, g_sem).wait()

    def wait_s(pp_v, rows_v, s_sem):
        pltpu.make_async_copy(rows_v, s_sh.at[pp_v.at[1]], s_sem).wait()

    fetch_idx(0, pp_a, w_a, i_a)
    gather(0, pp_a, w_a, rows_a, i_a, g_a)

    def pair(i, carry):
        @pl.when(i > 0)
        def _():
            wait_s(pp_b, rows_b, s_b)
        fetch_idx(2 * i + 1, pp_b, w_b, i_b)
        gather(2 * i + 1, pp_b, w_b, rows_b, i_b, g_b)
        wait_g(pp_a, rows_a, g_a)
        scale(w_a, rows_a)
        pltpu.async_copy(rows_a, s_sh.at[pp_a.at[1]], s_a, add=True)
        wait_g(pp_b, rows_b, g_b)
        scale(w_b, rows_b)
        pltpu.async_copy(rows_b, s_sh.at[pp_b.at[1]], s_b, add=True)
        wait_s(pp_a, rows_a, s_a)
        fetch_idx(2 * i + 2, pp_a, w_a, i_a)
        gather(2 * i + 2, pp_a, w_a, rows_a, i_a, g_a)
        return carry

    lax.fori_loop(0, (NCHUNK - 1) // 2, pair, 0)
    wait_s(pp_b, rows_b, s_b)
    wait_g(pp_a, rows_a, g_a)
    scale(w_a, rows_a)
    pltpu.sync_copy(rows_a, s_sh.at[pp_a.at[1]], add=True)
    plsc.subcore_barrier()
    pltpu.sync_copy(s_sh.at[pl.ds(r0, ROWS_PER_TILE)],
                    s_out.at[c].at[pl.ds(r0, ROWS_PER_TILE)])

    @pl.when(s == 0)
    def _out_tail():
        t0 = NS * ROWS_PER_TILE
        pltpu.sync_copy(s_sh.at[pl.ds(t0, TAIL_ROWS)],
                        s_out.at[c].at[pl.ds(t0, TAIL_ROWS)])


_sc_scatter = pl.kernel(
    _sc_body,
    out_type=jax.ShapeDtypeStruct((NC, N, D), jnp.float32),
    mesh=_MESH,
    scratch_types=[
        pltpu.VMEM((2, CHUNK), jnp.int32),
        pltpu.VMEM((CHUNK, 16), jnp.float32),
        pltpu.VMEM((CHUNK, D), jnp.float32),
        pltpu.VMEM((2, CHUNK), jnp.int32),
        pltpu.VMEM((CHUNK, 16), jnp.float32),
        pltpu.VMEM((CHUNK, D), jnp.float32),
        pltpu.VMEM_SHARED((N, D), jnp.float32),
        pltpu.SemaphoreType.DMA,
        pltpu.SemaphoreType.DMA,
        pltpu.SemaphoreType.DMA,
        pltpu.SemaphoreType.DMA,
        pltpu.SemaphoreType.DMA,
        pltpu.SemaphoreType.DMA,
    ],
)


def _sc_extras_body(dst_hbm, ext_hbm, z128_hbm, x_out,
                    dst_a, ext_a, rows_a, dst_b, ext_b, rows_b, x_sh,
                    s_a, s_b):
    c = lax.axis_index("c")
    s = lax.axis_index("s")
    wid = c * NS + s
    r0 = s * ROWS_PER_TILE
    pltpu.sync_copy(z128_hbm.at[pl.ds(r0, ROWS_PER_TILE)],
                    x_sh.at[pl.ds(r0, ROWS_PER_TILE)])

    @pl.when(s == 0)
    def _zero_tail():
        t0 = NS * ROWS_PER_TILE
        pltpu.sync_copy(z128_hbm.at[pl.ds(t0, TAIL_ROWS)],
                        x_sh.at[pl.ds(t0, TAIL_ROWS)])

    zv = jnp.zeros((16,), jnp.float32)

    def zrow(r, carry):
        for cb in range(D // 16):
            rows_a[r, pl.ds(cb * 16, 16)] = zv
            rows_b[r, pl.ds(cb * 16, 16)] = zv
        return carry

    lax.fori_loop(0, CHUNK, zrow, 0, unroll=4)
    plsc.subcore_barrier()

    base0 = wid * EWP

    def do_chunk(ci, dst_v, ext_v, rows_v, s_sem):
        base = base0 + ci * CHUNK
        pltpu.sync_copy(dst_hbm.at[pl.ds(base, CHUNK)], dst_v)
        pltpu.sync_copy(ext_hbm.at[pl.ds(base, CHUNK)], ext_v)

        def row_body(r, carry2):
            rows_v[r, pl.ds(0, 16)] = ext_v[r]
            return carry2

        lax.fori_loop(0, CHUNK, row_body, 0, unroll=8)
        pltpu.async_copy(rows_v, x_sh.at[dst_v], s_sem, add=True)

    def wait_s(dst_v, rows_v, s_sem):
        pltpu.make_async_copy(rows_v, x_sh.at[dst_v], s_sem).wait()

    def pair(i, carry):
        @pl.when(i > 0)
        def _():
            wait_s(dst_a, rows_a, s_a)
        do_chunk(2 * i, dst_a, ext_a, rows_a, s_a)

        @pl.when(i > 0)
        def _():
            wait_s(dst_b, rows_b, s_b)
        do_chunk(2 * i + 1, dst_b, ext_b, rows_b, s_b)
        return carry

    lax.fori_loop(0, (NCHUNK - 1) // 2, pair, 0)
    wait_s(dst_a, rows_a, s_a)
    do_chunk(NCHUNK - 1, dst_a, ext_a, rows_a, s_a)
    wait_s(dst_a, rows_a, s_a)
    wait_s(dst_b, rows_b, s_b)
    plsc.subcore_barrier()
    pltpu.sync_copy(x_sh.at[pl.ds(r0, ROWS_PER_TILE)],
                    x_out.at[c].at[pl.ds(r0, ROWS_PER_TILE)])

    @pl.when(s == 0)
    def _out_tail():
        t0 = NS * ROWS_PER_TILE
        pltpu.sync_copy(x_sh.at[pl.ds(t0, TAIL_ROWS)],
                        x_out.at[c].at[pl.ds(t0, TAIL_ROWS)])


_sc_extras = pl.kernel(
    _sc_extras_body,
    out_type=jax.ShapeDtypeStruct((NC, N, D), jnp.float32),
    mesh=_MESH,
    scratch_types=[
        pltpu.VMEM((CHUNK,), jnp.int32),
        pltpu.VMEM((CHUNK, 16), jnp.float32),
        pltpu.VMEM((CHUNK, D), jnp.float32),
        pltpu.VMEM((CHUNK,), jnp.int32),
        pltpu.VMEM((CHUNK, 16), jnp.float32),
        pltpu.VMEM((CHUNK, D), jnp.float32),
        pltpu.VMEM_SHARED((N, D), jnp.float32),
        pltpu.SemaphoreType.DMA,
        pltpu.SemaphoreType.DMA,
    ],
)


def _embed_body(an_ref, co_ref, wa_ref, ba_ref, wc_ref, bc_ref, wn_ref,
                bn_ref, h_ref):
    a = jnp.dot(an_ref[:], wa_ref[:], preferred_element_type=jnp.float32)
    a = a + ba_ref[:]
    c = jnp.dot(co_ref[:], wc_ref[:], preferred_element_type=jnp.float32)
    c = c + bc_ref[:]
    ac = jnp.concatenate([a, c], axis=1)
    h_ref[:] = jnp.dot(ac, wn_ref[:], preferred_element_type=jnp.float32) + bn_ref[:]


def _embed(atomic_num, coord, W_atom, b_atom, W_coord, b_coord, W_node, b_node):
    return pl.pallas_call(
        _embed_body,
        out_shape=jax.ShapeDtypeStruct((N, D), jnp.float32),
    )(atomic_num, coord, W_atom, b_atom.reshape(1, D),
      W_coord, b_coord.reshape(1, D), W_node, b_node.reshape(1, D))


_EB = EWP


def _edgew_body(len_ref, e0_ref, e1_ref, w0_ref, w1_ref, ext_ref):
    ln = len_ref[:]
    lg = jnp.log(ln)
    valid = lax.broadcasted_iota(jnp.int32, (_EB, 1), 0) < EW
    w0 = jnp.where(valid, jnp.exp(lg * e0_ref[0, 0]), 0.0)
    w1 = jnp.where(valid, jnp.exp(lg * e1_ref[0, 0]), 0.0)
    w0_ref[:] = jnp.broadcast_to(w0, (_EB, 16))
    w1_ref[:] = jnp.broadcast_to(w1, (_EB, 16))
    z = jnp.zeros((_EB, 12), jnp.float32)
    ext_ref[:] = jnp.concatenate([w0, w0 * ln, w1, w1 * ln, z], axis=1)


def _edgew(length, exp0, exp1):
    nb = EPAD // _EB
    return pl.pallas_call(
        _edgew_body,
        grid=(nb,),
        in_specs=[
            pl.BlockSpec((_EB, 1), lambda i: (i, 0)),
            pl.BlockSpec((1, 1), lambda i: (0, 0)),
            pl.BlockSpec((1, 1), lambda i: (0, 0)),
        ],
        out_specs=[
            pl.BlockSpec((_EB, 16), lambda i: (i, 0)),
            pl.BlockSpec((_EB, 16), lambda i: (i, 0)),
            pl.BlockSpec((_EB, 16), lambda i: (i, 0)),
        ],
        out_shape=[
            jax.ShapeDtypeStruct((EPAD, 16), jnp.float32),
            jax.ShapeDtypeStruct((EPAD, 16), jnp.float32),
            jax.ShapeDtypeStruct((EPAD, 16), jnp.float32),
        ],
    )(length, exp0.reshape(1, 1), exp1.reshape(1, 1))


_NB = 1000
_NGRID = N // _NB


def _conv_tail(s2, x2, h, agg_w, agg_b, w_edge, b_edge, wv, bv, wg, bg, eps,
               s_cols):
    S = s2[0] + s2[1]
    X = x2[0] + x2[1]
    ws = agg_w[0:D, :]
    we = agg_w[D:2 * D, :]
    wd = agg_w[2 * D:3 * D, :]
    u = jnp.dot(w_edge, we, preferred_element_type=jnp.float32)
    ce = jnp.dot(b_edge, we, preferred_element_type=jnp.float32) + agg_b
    s0 = X[:, s_cols[0]:s_cols[0] + 1]
    s1 = X[:, s_cols[1]:s_cols[1] + 1]
    agg = (jnp.dot(S, ws, preferred_element_type=jnp.float32)
           + s1 * u + s0 * ce
           + jnp.dot(s0 * h, wd, preferred_element_type=jnp.float32))
    rst = (1.0 + eps) * h + agg
    gv = jnp.dot(rst, wv, preferred_element_type=jnp.float32) + bv
    gg = jnp.dot(rst, wg, preferred_element_type=jnp.float32) + bg
    return gv * jax.nn.sigmoid(gg)


def _combine_mid_body(s2_ref, x2_ref, h_ref, aggw_ref, aggb_ref, wed_ref,
                      bed_ref, wv_ref, bv_ref, wg_ref, bg_ref, eps_ref,
                      h1_ref):
    h1_ref[:] = _conv_tail(
        s2_ref[:], x2_ref[:], h_ref[:], aggw_ref[:], aggb_ref[:], wed_ref[:],
        bed_ref[:], wv_ref[:], bv_ref[:], wg_ref[:], bg_ref[:],
        eps_ref[0, 0], (0, 1))


def _combine_mid(s2, x2, h, agg_w, agg_b, w_edge, b_edge, wv, bv, wg, bg, eps):
    full = lambda shape: pl.BlockSpec(shape, lambda i: tuple(0 for _ in shape))
    return pl.pallas_call(
        _combine_mid_body,
        grid=(_NGRID,),
        in_specs=[
            pl.BlockSpec((NC, _NB, D), lambda i: (0, i, 0)),
            pl.BlockSpec((NC, _NB, 8), lambda i: (0, i, 0)),
            pl.BlockSpec((_NB, D), lambda i: (i, 0)),
            full((3 * D, D)), full((1, D)), full((1, D)), full((1, D)),
            full((D, D)), full((1, D)), full((D, D)), full((1, D)),
            full((1, 1)),
        ],
        out_specs=pl.BlockSpec((_NB, D), lambda i: (i, 0)),
        out_shape=jax.ShapeDtypeStruct((N, D), jnp.float32),
    )(s2, x2, h, agg_w, agg_b.reshape(1, D), w_edge, b_edge.reshape(1, D),
      wv, bv.reshape(1, D), wg, bg.reshape(1, D), eps.reshape(1, 1))


def _combine_last_body(s2_ref, x2_ref, h_ref, aggw_ref, aggb_ref, wed_ref,
                       bed_ref, wv_ref, bv_ref, wg_ref, bg_ref, eps_ref,
                       mask_ref, wm_ref, bm_ref, out_ref, feat_ref):
    h2 = _conv_tail(
        s2_ref[:], x2_ref[:], h_ref[:], aggw_ref[:], aggb_ref[:], wed_ref[:],
        bed_ref[:], wv_ref[:], bv_ref[:], wg_ref[:], bg_ref[:],
        eps_ref[0, 0], (2, 3))
    h2 = jnp.where(mask_ref[:] == 0, 0.0, h2)
    part = jnp.sum(h2, axis=0, keepdims=True) * (1.0 / N)

    @pl.when(pl.program_id(0) == 0)
    def _init():
        feat_ref[:] = jnp.zeros_like(feat_ref)

    feat_ref[:] += part

    @pl.when(pl.program_id(0) == _NGRID - 1)
    def _head():
        out_ref[:] = jax.nn.sigmoid(
            jnp.dot(feat_ref[:], wm_ref[:],
                    preferred_element_type=jnp.float32) + bm_ref[:])


def _combine_last(s2, x2, h, agg_w, agg_b, w_edge, b_edge, wv, bv, wg, bg,
                  eps, abs_mask, w_mlp, b_mlp):
    full = lambda shape: pl.BlockSpec(shape, lambda i: tuple(0 for _ in shape))
    out, _ = pl.pallas_call(
        _combine_last_body,
        grid=(_NGRID,),
        in_specs=[
            pl.BlockSpec((NC, _NB, D), lambda i: (0, i, 0)),
            pl.BlockSpec((NC, _NB, 8), lambda i: (0, i, 0)),
            pl.BlockSpec((_NB, D), lambda i: (i, 0)),
            full((3 * D, D)), full((1, D)), full((1, D)), full((1, D)),
            full((D, D)), full((1, D)), full((D, D)), full((1, D)),
            full((1, 1)),
            pl.BlockSpec((_NB, 1), lambda i: (i, 0)),
            full((D, OUT_DIM)), full((1, OUT_DIM)),
        ],
        out_specs=[pl.BlockSpec((1, OUT_DIM), lambda i: (0, 0)),
                   pl.BlockSpec((1, D), lambda i: (0, 0))],
        out_shape=[jax.ShapeDtypeStruct((1, OUT_DIM), jnp.float32),
                   jax.ShapeDtypeStruct((1, D), jnp.float32)],
    )(s2, x2, h, agg_w, agg_b.reshape(1, D), w_edge, b_edge.reshape(1, D),
      wv, bv.reshape(1, D), wg, bg.reshape(1, D), eps.reshape(1, 1),
      abs_mask.reshape(N, 1), w_mlp, b_mlp.reshape(1, OUT_DIM))
    return out


def kernel(atomic_num, coord, length, abs_mask, edge_index, W_atom, b_atom,
           W_coord, b_coord, W_node, b_node, W_edge, b_edge, agg_W0, agg_b0,
           glu_Wv0, glu_bv0, glu_Wg0, glu_bg0, exp0, eps0, agg_W1, agg_b1,
           glu_Wv1, glu_bv1, glu_Wg1, glu_bg1, exp1, eps1, W_mlp, b_mlp):
    pad2 = jnp.zeros((2, NW, EWP - EW), jnp.int32)
    pad2 = pad2 + (jnp.arange(EWP - EW, dtype=jnp.int32) % N)[None, None, :]
    ei = jnp.concatenate([edge_index.reshape(2, NW, EW), pad2], axis=2)
    ei = ei.reshape(2, EPAD)
    src = ei[0]
    dst = ei[1]
    p3 = ei.reshape(2, NW * NCHUNK, CHUNK).transpose(1, 0, 2)
    lenp = jnp.concatenate(
        [length.reshape(NW, EW), jnp.ones((NW, EWP - EW), jnp.float32)],
        axis=1).reshape(EPAD, 1)
    h0 = _embed(atomic_num, coord, W_atom, b_atom, W_coord, b_coord,
                W_node, b_node)
    w0b, w1b, wext = _edgew(lenp, exp0, exp1)
    z128 = jnp.zeros((N, D), jnp.float32)

    s_l0 = _sc_scatter(h0, p3, w0b, z128)
    x_l0 = _sc_extras(dst, wext, z128)
    x8 = x_l0[:, :, :8]
    h1 = _combine_mid(s_l0, x8, h0, agg_W0, agg_b0, W_edge, b_edge,
                      glu_Wv0, glu_bv0, glu_Wg0, glu_bg0, eps0)
    s_l1 = _sc_scatter(h1, p3, w1b, z128)
    return _combine_last(s_l1, x8, h1, agg_W1, agg_b1, W_edge, b_edge,
                         glu_Wv1, glu_bv1, glu_Wg1, glu_bg1, eps1, abs_mask,
                         W_mlp, b_mlp)
